# baseline (speedup 1.0000x reference)
import functools

import jax
import jax.numpy as jnp
from jax import lax
from jax.experimental import pallas as pl
from jax.experimental.pallas import tpu as pltpu

N = 10000
FEAT = 1024
D = 200
DP = 208
R = 200
E = 160000
EH = E // 2
NQ = 40000
NQH = NQ // 2


def _pad_d(a, axes):
    pads = [(0, 0)] * a.ndim
    for ax in axes:
        pads[ax] = (0, DP - a.shape[ax])
    return jnp.pad(a, pads)


def _mm_bias_body(x_ref, w_ref, b_ref, o_ref):
    o_ref[...] = jnp.dot(x_ref[...], w_ref[...],
                         preferred_element_type=jnp.float32) + b_ref[...]


def _tc_matmul_bias(x, w, b, bm):
    m, k = x.shape
    n = w.shape[1]
    return pl.pallas_call(
        _mm_bias_body,
        grid=(m // bm,),
        in_specs=[
            pl.BlockSpec((bm, k), lambda i: (i, 0)),
            pl.BlockSpec((k, n), lambda i: (0, 0)),
            pl.BlockSpec((1, n), lambda i: (0, 0)),
        ],
        out_specs=pl.BlockSpec((bm, n), lambda i: (i, 0)),
        out_shape=jax.ShapeDtypeStruct((m, n), jnp.float32),
    )(x, w, b.reshape(1, n))


def _qw_body(q_ref, w_ref, qw_ref, r_ref):
    q = q_ref[...]
    qw_ref[...] = jnp.dot(q, w_ref[...], preferred_element_type=jnp.float32)
    r_ref[...] = jnp.sum(q, axis=-1, keepdims=True)


def _tc_qw(q_emb, w_q, bm):
    m = q_emb.shape[0]
    qw, r = pl.pallas_call(
        _qw_body,
        grid=(m // bm,),
        in_specs=[
            pl.BlockSpec((bm, DP), lambda i: (i, 0)),
            pl.BlockSpec((DP, DP), lambda i: (0, 0)),
        ],
        out_specs=[
            pl.BlockSpec((bm, DP), lambda i: (i, 0)),
            pl.BlockSpec((bm, 1), lambda i: (i, 0)),
        ],
        out_shape=[
            jax.ShapeDtypeStruct((m, DP), jnp.float32),
            jax.ShapeDtypeStruct((m, 1), jnp.float32),
        ],
    )(q_emb, w_q)
    return qw, r[:, 0]


def _combine_body(ain_ref, aout_ref, h_ref, wi_ref, wo_ref, wl_ref, lr_ref,
                  o_ref):
    acc = jnp.dot(ain_ref[...], wi_ref[...], preferred_element_type=jnp.float32)
    acc += jnp.dot(aout_ref[...], wo_ref[...], preferred_element_type=jnp.float32)
    acc += jnp.dot(h_ref[...] * lr_ref[...], wl_ref[...],
                   preferred_element_type=jnp.float32)
    o_ref[...] = jnp.tanh(acc * (1.0 / 3.0))


def _tc_combine(a_in, a_out, h, w_in, w_out, w_loop, loop_rel, bm):
    m = h.shape[0]
    return pl.pallas_call(
        _combine_body,
        grid=(m // bm,),
        in_specs=[
            pl.BlockSpec((bm, DP), lambda i: (i, 0)),
            pl.BlockSpec((bm, DP), lambda i: (i, 0)),
            pl.BlockSpec((bm, DP), lambda i: (i, 0)),
            pl.BlockSpec((DP, DP), lambda i: (0, 0)),
            pl.BlockSpec((DP, DP), lambda i: (0, 0)),
            pl.BlockSpec((DP, DP), lambda i: (0, 0)),
            pl.BlockSpec((1, DP), lambda i: (0, 0)),
        ],
        out_specs=pl.BlockSpec((bm, DP), lambda i: (i, 0)),
        out_shape=jax.ShapeDtypeStruct((m, DP), jnp.float32),
    )(a_in, a_out, h, w_in, w_out, w_loop, loop_rel)


def _relmm_body(rf_ref, w_ref, o_ref):
    o_ref[...] = jnp.dot(rf_ref[...], w_ref[...],
                         preferred_element_type=jnp.float32)


def _tc_relmm(rel_full, w_rel):
    return pl.pallas_call(
        _relmm_body,
        in_specs=[pl.BlockSpec((DP, DP), lambda: (0, 0)),
                  pl.BlockSpec((DP, DP), lambda: (0, 0))],
        out_specs=pl.BlockSpec((DP, DP), lambda: (0, 0)),
        out_shape=jax.ShapeDtypeStruct((DP, DP), jnp.float32),
        grid=(),
    )(rel_full, w_rel)


def _sparse_norm(src, dst):
    deg = jnp.zeros((2, N), jnp.float32)
    deg = deg.at[jnp.arange(2)[:, None], src].add(1.0)
    deg_inv = jnp.where(deg > 0, lax.rsqrt(deg), 0.0)
    take = jnp.take_along_axis
    return take(deg_inv, src, axis=1) * take(deg_inv, dst, axis=1)


def kernel(x, edge_index, edge_type, qualifier_index, rel_embs, fr_W, fr_b,
           w_in_0, w_out_0, w_loop_0, w_rel_0, w_q_0, loop_rel_0,
           w_in_1, w_out_1, w_loop_1, w_rel_1, w_q_1, loop_rel_1):
    src = edge_index[0].reshape(2, EH)
    dst = edge_index[1].reshape(2, EH)
    etype = edge_type.reshape(2, EH)
    q_rel = qualifier_index[0].reshape(2, NQH)
    q_ent = qualifier_index[1].reshape(2, NQH)
    q_edge = qualifier_index[2].reshape(2, NQH)

    norm = _sparse_norm(src, dst)
    h2 = jnp.arange(2)[:, None]
    src_q = jnp.take_along_axis(src, q_edge, axis=1)
    dst_q = jnp.take_along_axis(dst, q_edge, axis=1)
    etype_q = jnp.take_along_axis(etype, q_edge, axis=1)
    norm_q = jnp.take_along_axis(norm, q_edge, axis=1)

    h = _tc_matmul_bias(x, _pad_d(fr_W, (1,)), _pad_d(fr_b, (0,)), bm=1000)

    rel = _pad_d(rel_embs, (1,))
    layers = [
        (w_in_0, w_out_0, w_loop_0, w_rel_0, w_q_0, loop_rel_0),
        (w_in_1, w_out_1, w_loop_1, w_rel_1, w_q_1, loop_rel_1),
    ]
    for w_in, w_out, w_loop, w_rel, w_q, loop_rel in layers:
        w_in, w_out, w_loop, w_rel, w_q = (
            _pad_d(w, (0, 1)) for w in (w_in, w_out, w_loop, w_rel, w_q))
        loop_rel = _pad_d(loop_rel, (1,))
        rel_full = jnp.zeros((DP, DP), jnp.float32)
        rel_full = rel_full.at[:R].set(rel).at[R].set(loop_rel[0])
        cq = jnp.sum(w_q[:D], axis=0)
        relq = rel_full * cq[None, :]

        q_emb = h[q_ent] * rel_full[q_rel]
        qw, r = _tc_qw(q_emb.reshape(NQ, DP), w_q, bm=1000)
        qw = qw.reshape(2, NQH, DP)
        r = r.reshape(2, NQH)
        s = jnp.zeros((2, EH), jnp.float32).at[h2, q_edge].add(r)
        s_q = jnp.take_along_axis(s, q_edge, axis=1)

        scale_e = norm * (s == 0.0)
        m_e = h[src] * relq[etype] * scale_e[..., None]
        scale_q = norm_q * (s_q != 0.0)
        m_q = h[src_q] * rel_full[etype_q] * qw * scale_q[..., None]
        acc = jnp.zeros((2, N, DP), jnp.float32)
        acc = acc.at[h2, dst].add(m_e)
        acc = acc.at[h2, dst_q].add(m_q)

        h = _tc_combine(acc[0], acc[1], h, w_in, w_out, w_loop, loop_rel,
                        bm=1000)
        rel = _tc_relmm(rel_full, w_rel)[:R]

    return h[:, :D], rel[:R, :D]


# speedup vs baseline: 1.3101x; 1.3101x over previous
import functools

import jax
import jax.numpy as jnp
from jax import lax
from jax.experimental import pallas as pl
from jax.experimental.pallas import tpu as pltpu
from jax.experimental.pallas import tpu_sc as plsc

N = 10000
FEAT = 1024
D = 200
DP = 208
R = 200
E = 160000
EH = E // 2
NQH = 40000 // 2

NS = 16
LANES = 16
BS = 32

NR = 10048
DUMMY = 10016
EHP = 81920
EPT = EHP // NS
EBLK = 1280
NQHP = 20480
QPT = NQHP // NS
ART = NR // NS
CW_A = 112
CW_B = DP - CW_A

_MESH = plsc.VectorSubcoreMesh(core_axis_name="c", subcore_axis_name="s")
_SC_PARAMS = pltpu.CompilerParams(use_tc_tiling_on_sc=False,
                                  needs_layout_passes=False)


def _pad_d(a, axes):
    pads = [(0, 0)] * a.ndim
    for ax in axes:
        pads[ax] = (0, DP - a.shape[ax])
    return jnp.pad(a, pads)


def _mm_bias_body(x_ref, w_ref, b_ref, o_ref):
    o_ref[...] = jnp.dot(x_ref[...], w_ref[...],
                         preferred_element_type=jnp.float32) + b_ref[...]


def _tc_matmul_bias(x, w, b, bm):
    m, k = x.shape
    n = w.shape[1]
    return pl.pallas_call(
        _mm_bias_body,
        grid=(m // bm,),
        in_specs=[
            pl.BlockSpec((bm, k), lambda i: (i, 0)),
            pl.BlockSpec((k, n), lambda i: (0, 0)),
            pl.BlockSpec((1, n), lambda i: (0, 0)),
        ],
        out_specs=pl.BlockSpec((bm, n), lambda i: (i, 0)),
        out_shape=jax.ShapeDtypeStruct((m, n), jnp.float32),
    )(x, w, b.reshape(1, n))


def _qw_body(q_ref, w_ref, o_ref):
    o_ref[...] = jnp.dot(q_ref[...], w_ref[...],
                         preferred_element_type=jnp.float32)


def _tc_qw(q_emb, w_q, bm):
    m = q_emb.shape[0]
    return pl.pallas_call(
        _qw_body,
        grid=(m // bm,),
        in_specs=[
            pl.BlockSpec((bm, DP), lambda i: (i, 0)),
            pl.BlockSpec((DP, DP), lambda i: (0, 0)),
        ],
        out_specs=pl.BlockSpec((bm, DP), lambda i: (i, 0)),
        out_shape=jax.ShapeDtypeStruct((m, DP), jnp.float32),
    )(q_emb, w_q)


def _relprep_body(rf_ref, wq_ref, wr_ref, relq_ref, reln_ref):
    wq = wq_ref[...]
    cq = jnp.sum(wq, axis=0, keepdims=True)
    relq_ref[...] = rf_ref[...] * cq
    reln_ref[...] = jnp.dot(rf_ref[...], wr_ref[...],
                            preferred_element_type=jnp.float32)


def _tc_relprep(rel_full, w_q, w_rel):
    return pl.pallas_call(
        _relprep_body,
        in_specs=[pl.BlockSpec((DP, DP), lambda: (0, 0))] * 3,
        out_specs=[pl.BlockSpec((DP, DP), lambda: (0, 0))] * 2,
        out_shape=[jax.ShapeDtypeStruct((DP, DP), jnp.float32)] * 2,
        grid=(),
    )(rel_full, w_q, w_rel)


def _combine_body(aina_ref, ainb_ref, aouta_ref, aoutb_ref, h_ref,
                  wi_ref, wo_ref, wl_ref, lr_ref, o_ref):
    f32 = jnp.float32
    acc = jnp.dot(aina_ref[...], wi_ref[:CW_A, :], preferred_element_type=f32)
    acc += jnp.dot(ainb_ref[...], wi_ref[CW_A:, :], preferred_element_type=f32)
    acc += jnp.dot(aouta_ref[...], wo_ref[:CW_A, :], preferred_element_type=f32)
    acc += jnp.dot(aoutb_ref[...], wo_ref[CW_A:, :], preferred_element_type=f32)
    acc += jnp.dot(h_ref[...] * lr_ref[...], wl_ref[...],
                   preferred_element_type=f32)
    o_ref[...] = jnp.tanh(acc * (1.0 / 3.0))


def _tc_combine(aina, ainb, aouta, aoutb, h, w_in, w_out, w_loop, loop_rel,
                bm):
    m = h.shape[0]
    return pl.pallas_call(
        _combine_body,
        grid=(m // bm,),
        in_specs=[
            pl.BlockSpec((bm, CW_A), lambda i: (i, 0)),
            pl.BlockSpec((bm, CW_B), lambda i: (i, 0)),
            pl.BlockSpec((bm, CW_A), lambda i: (i, 0)),
            pl.BlockSpec((bm, CW_B), lambda i: (i, 0)),
            pl.BlockSpec((bm, DP), lambda i: (i, 0)),
            pl.BlockSpec((DP, DP), lambda i: (0, 0)),
            pl.BlockSpec((DP, DP), lambda i: (0, 0)),
            pl.BlockSpec((DP, DP), lambda i: (0, 0)),
            pl.BlockSpec((1, DP), lambda i: (0, 0)),
        ],
        out_specs=pl.BlockSpec((bm, DP), lambda i: (i, 0)),
        out_shape=jax.ShapeDtypeStruct((m, DP), jnp.float32),
    )(aina, ainb, aouta, aoutb, h, w_in, w_out, w_loop, loop_rel)


def _rsqrt16(d):
    i = plsc.bitcast(d, jnp.int32)
    i = jnp.int32(0x5F3759DF) - lax.shift_right_logical(i, 1)
    y = plsc.bitcast(i, jnp.float32)
    for _ in range(3):
        y = y * (1.5 - 0.5 * d * y * y)
    return jnp.where(d > 0, y, 0.0)


def _fill(ref, n, value):
    v = jnp.full((LANES,), value, ref.dtype)

    def body(i, _):
        ref[pl.ds(i * LANES, LANES)] = v
        return 0

    lax.fori_loop(0, n // LANES, body, 0)


@functools.partial(
    pl.kernel,
    out_type=[
        jax.ShapeDtypeStruct((2 * EHP,), jnp.float32),
        jax.ShapeDtypeStruct((2 * NQHP,), jnp.int32),
        jax.ShapeDtypeStruct((2 * NQHP,), jnp.int32),
        jax.ShapeDtypeStruct((2 * NQHP,), jnp.int32),
        jax.ShapeDtypeStruct((2 * NQHP,), jnp.float32),
    ],
    mesh=_MESH,
    compiler_params=_SC_PARAMS,
    scratch_types=[
        pltpu.VMEM_SHARED((NR,), jnp.float32),
        pltpu.VMEM((EPT,), jnp.int32),
        pltpu.VMEM((EPT,), jnp.int32),
        pltpu.VMEM((EPT,), jnp.float32),
        pltpu.VMEM((640,), jnp.float32),
        pltpu.VMEM((NR,), jnp.float32),
        pltpu.VMEM((EHP,), jnp.int32),
        pltpu.VMEM((QPT,), jnp.int32),
        pltpu.VMEM((QPT,), jnp.int32),
        pltpu.VMEM((QPT,), jnp.int32),
        pltpu.VMEM((QPT,), jnp.float32),
    ],
)
def _sc_prep(src_hbm, dst_hbm, et_hbm, qedge_hbm,
             norm_out, srcq_out, dstq_out, etq_out, normq_out,
             deg_sh, src_v, dst_v, val_v, zb_v, dinv_v, tbl_v,
             qe_v, sq_v, dq_v, nq_v):
    c = lax.axis_index("c")
    t = lax.axis_index("s")
    ebase = c * EHP + t * EPT
    qbase = c * NQHP + t * QPT

    pltpu.sync_copy(src_hbm.at[pl.ds(ebase, EPT)], src_v)
    pltpu.sync_copy(dst_hbm.at[pl.ds(ebase, EPT)], dst_v)

    _fill(zb_v, 640, 0.0)
    zbase = jnp.minimum(t * 640, NR - 640)
    pltpu.sync_copy(zb_v, deg_sh.at[pl.ds(zbase, 640)])
    _fill(val_v, EPT, 1.0)
    plsc.subcore_barrier()

    pltpu.sync_copy(val_v, deg_sh.at[src_v], add=True)
    plsc.subcore_barrier()

    pltpu.sync_copy(deg_sh, dinv_v)

    def inv_body(i, _):
        d = dinv_v[pl.ds(i * LANES, LANES)]
        dinv_v[pl.ds(i * LANES, LANES)] = _rsqrt16(d)
        return 0

    lax.fori_loop(0, NR // LANES, inv_body, 0)

    def norm_body(i, _):
        sl = pl.ds(i * LANES, LANES)
        a = plsc.load_gather(dinv_v, [src_v[sl]])
        b = plsc.load_gather(dinv_v, [dst_v[sl]])
        val_v[sl] = a * b
        return 0

    lax.fori_loop(0, EPT // LANES, norm_body, 0)
    pltpu.sync_copy(val_v, norm_out.at[pl.ds(ebase, EPT)])

    pltpu.sync_copy(qedge_hbm.at[pl.ds(qbase, QPT)], qe_v)

    def gather_q(table_hbm, out_v):
        pltpu.sync_copy(table_hbm.at[pl.ds(c * EHP, EHP)], tbl_v)

        def body(i, _):
            sl = pl.ds(i * LANES, LANES)
            out_v[sl] = plsc.load_gather(tbl_v, [qe_v[sl]])
            return 0

        lax.fori_loop(0, QPT // LANES, body, 0)

    gather_q(src_hbm, sq_v)
    pltpu.sync_copy(sq_v, srcq_out.at[pl.ds(qbase, QPT)])
    gather_q(dst_hbm, dq_v)
    pltpu.sync_copy(dq_v, dstq_out.at[pl.ds(qbase, QPT)])

    def normq_body(i, _):
        sl = pl.ds(i * LANES, LANES)
        a = plsc.load_gather(dinv_v, [sq_v[sl]])
        b = plsc.load_gather(dinv_v, [dq_v[sl]])
        nq_v[sl] = a * b
        return 0

    lax.fori_loop(0, QPT // LANES, normq_body, 0)
    pltpu.sync_copy(nq_v, normq_out.at[pl.ds(qbase, QPT)])

    gather_q(et_hbm, dq_v)
    pltpu.sync_copy(dq_v, etq_out.at[pl.ds(qbase, QPT)])


@functools.partial(
    pl.kernel,
    out_type=[
        jax.ShapeDtypeStruct((2 * NQHP, DP), jnp.float32),
        jax.ShapeDtypeStruct((2 * EHP,), jnp.float32),
        jax.ShapeDtypeStruct((2 * EHP,), jnp.float32),
    ],
    mesh=_MESH,
    compiler_params=_SC_PARAMS,
    scratch_types=[
        pltpu.VMEM_SHARED((EHP,), jnp.float32),
        pltpu.VMEM((QPT,), jnp.int32),
        pltpu.VMEM((QPT,), jnp.int32),
        pltpu.VMEM((QPT,), jnp.int32),
        pltpu.VMEM((QPT,), jnp.float32),
        pltpu.VMEM((DP, DP), jnp.float32),
        pltpu.VMEM((BS, DP), jnp.float32),
        pltpu.VMEM((BS, DP), jnp.float32),
        pltpu.VMEM((BS, DP), jnp.float32),
        pltpu.VMEM((BS, DP), jnp.float32),
        pltpu.VMEM((EPT,), jnp.float32),
        pltpu.VMEM((EPT,), jnp.float32),
        pltpu.VMEM((640,), jnp.float32),
        pltpu.SemaphoreType.DMA,
        pltpu.SemaphoreType.DMA,
        pltpu.SemaphoreType.DMA,
        pltpu.SemaphoreType.DMA,
    ],
)
def _sc_qualprep(h_hbm, relf_hbm, qent_hbm, qrel_hbm, qedge_hbm, norm_hbm,
                 qemb_out, s_out, scale_out,
                 s_sh, qent_v, qrel_v, qe_v, r_v, relf_t,
                 hbuf0, hbuf1, qbuf0, qbuf1, nrm_v, sl_v, zb_v,
                 sg0, sg1, sw0, sw1):
    c = lax.axis_index("c")
    t = lax.axis_index("s")
    qbase = c * NQHP + t * QPT
    ebase = c * EHP + t * EPT

    pltpu.sync_copy(qent_hbm.at[pl.ds(qbase, QPT)], qent_v)
    pltpu.sync_copy(qrel_hbm.at[pl.ds(qbase, QPT)], qrel_v)
    pltpu.sync_copy(qedge_hbm.at[pl.ds(qbase, QPT)], qe_v)
    pltpu.sync_copy(relf_hbm, relf_t)

    _fill(zb_v, 640, 0.0)
    for i in range(EPT // 640):
        pltpu.sync_copy(zb_v, s_sh.at[pl.ds(t * EPT + i * 640, 640)])
    plsc.subcore_barrier()

    j16 = lax.iota(jnp.int32, LANES)

    def compute(qb, hbuf, qbuf):
        for k in range(BS // LANES):
            jk = j16 + k * LANES
            qrel16 = qrel_v[pl.ds(qb + k * LANES, LANES)]

            def f_body(f, racc):
                f16 = jnp.full((LANES,), f, jnp.int32)
                hv = plsc.load_gather(hbuf, [jk, f16])
                rv = plsc.load_gather(relf_t, [qrel16, f16])
                q = hv * rv
                plsc.store_scatter(qbuf, [jk, f16], q)
                return racc + q

            racc = lax.fori_loop(0, DP, f_body,
                                 jnp.zeros((LANES,), jnp.float32), unroll=8)
            r_v[pl.ds(qb + k * LANES, LANES)] = racc

    def pair_body(p, _):
        b0 = p * 2 * BS
        b1 = b0 + BS
        g0 = pltpu.async_copy(h_hbm.at[qent_v.at[pl.ds(b0, BS)]], hbuf0, sg0)
        g1 = pltpu.async_copy(h_hbm.at[qent_v.at[pl.ds(b1, BS)]], hbuf1, sg1)
        g0.wait()
        compute(b0, hbuf0, qbuf0)
        w0 = pltpu.async_copy(qbuf0, qemb_out.at[pl.ds(qbase + b0, BS)], sw0)
        g1.wait()
        compute(b1, hbuf1, qbuf1)
        w1 = pltpu.async_copy(qbuf1, qemb_out.at[pl.ds(qbase + b1, BS)], sw1)
        w0.wait()
        w1.wait()
        return 0

    lax.fori_loop(0, QPT // (2 * BS), pair_body, 0)

    pltpu.sync_copy(r_v, s_sh.at[qe_v], add=True)
    plsc.subcore_barrier()
    pltpu.sync_copy(s_sh.at[pl.ds(t * EPT, EPT)], sl_v)
    pltpu.sync_copy(sl_v, s_out.at[pl.ds(ebase, EPT)])
    pltpu.sync_copy(norm_hbm.at[pl.ds(ebase, EPT)], nrm_v)

    def scale_body(i, _):
        sl = pl.ds(i * LANES, LANES)
        keep = sl_v[sl] == 0.0
        nrm_v[sl] = nrm_v[sl] * jnp.where(keep, 1.0, 0.0)
        return 0

    lax.fori_loop(0, EPT // LANES, scale_body, 0)
    pltpu.sync_copy(nrm_v, scale_out.at[pl.ds(ebase, EPT)])


def _make_sc_accum(cw):
    @functools.partial(
        pl.kernel,
        out_type=jax.ShapeDtypeStruct((2 * NR, cw), jnp.float32),
        mesh=_MESH,
        compiler_params=_SC_PARAMS,
        scratch_types=[
            pltpu.VMEM_SHARED((NR, cw), jnp.float32),
            pltpu.VMEM((DP, cw), jnp.float32),
            pltpu.VMEM((EBLK,), jnp.int32),
            pltpu.VMEM((EBLK,), jnp.int32),
            pltpu.VMEM((EBLK,), jnp.int32),
            pltpu.VMEM((EBLK,), jnp.float32),
            pltpu.VMEM((BS, cw), jnp.float32),
            pltpu.VMEM((BS, cw), jnp.float32),
            pltpu.VMEM((BS, cw), jnp.float32),
            pltpu.VMEM((BS, cw), jnp.float32),
            pltpu.VMEM((BS, cw), jnp.float32),
            pltpu.VMEM((BS, cw), jnp.float32),
            pltpu.VMEM((BS,), jnp.int32),
            pltpu.VMEM((BS,), jnp.int32),
            pltpu.VMEM((4, cw), jnp.float32),
            pltpu.VMEM((EBLK,), jnp.float32),
            pltpu.SemaphoreType.DMA,
            pltpu.SemaphoreType.DMA,
            pltpu.SemaphoreType.DMA,
            pltpu.SemaphoreType.DMA,
            pltpu.SemaphoreType.DMA,
            pltpu.SemaphoreType.DMA,
        ],
    )
    def _accum(h_hbm, relq_hbm, relf_hbm, dst_hbm, et_hbm, src_hbm,
               scale_hbm, s_hbm, srcq_hbm, dstq_hbm, etq_hbm, normq_hbm,
               qedge_hbm, qw_hbm,
               a_out,
               acc_sh, tbl_t, src_v, dst_v, et_v, scale_v,
               hbuf0, hbuf1, mbuf0, mbuf1, qwbuf0, qwbuf1,
               idx0, idx1, zrows, aux_v,
               sg0, sg1, ss0, ss1, sq0, sq1):
        c = lax.axis_index("c")
        t = lax.axis_index("s")
        ebase = c * EHP + t * EPT
        qbase = c * NQHP + t * QPT
        arow0 = t * ART

        for i in range(4):
            for k in range(cw // LANES):
                zrows[i, pl.ds(k * LANES, LANES)] = jnp.zeros((LANES,),
                                                              jnp.float32)

        def zero_body(i, _):
            pltpu.sync_copy(zrows, acc_sh.at[pl.ds(arow0 + i * 4, 4)])
            return 0

        lax.fori_loop(0, ART // 4, zero_body, 0)
        pltpu.sync_copy(relq_hbm, tbl_t)
        plsc.subcore_barrier()

        j16 = lax.iota(jnp.int32, LANES)

        def compute_edge(b, hbuf, mbuf):
            for k in range(BS // LANES):
                jk = j16 + k * LANES
                sc16 = scale_v[pl.ds(b + k * LANES, LANES)]
                et16 = et_v[pl.ds(b + k * LANES, LANES)]

                def f_body(f, _):
                    f16 = jnp.full((LANES,), f, jnp.int32)
                    hv = plsc.load_gather(hbuf, [jk, f16])
                    rv = plsc.load_gather(tbl_t, [et16, f16])
                    plsc.store_scatter(mbuf, [jk, f16], hv * rv * sc16)
                    return 0

                lax.fori_loop(0, cw, f_body, 0, unroll=8)

        def blk_body(blk, _):
            bbase = ebase + blk * EBLK
            pltpu.sync_copy(src_hbm.at[pl.ds(bbase, EBLK)], src_v)
            pltpu.sync_copy(dst_hbm.at[pl.ds(bbase, EBLK)], dst_v)
            pltpu.sync_copy(et_hbm.at[pl.ds(bbase, EBLK)], et_v)
            pltpu.sync_copy(scale_hbm.at[pl.ds(bbase, EBLK)], scale_v)

            def pair_body(p, _):
                b0 = p * 2 * BS
                b1 = b0 + BS
                g0 = pltpu.async_copy(h_hbm.at[src_v.at[pl.ds(b0, BS)]],
                                      hbuf0, sg0)
                g1 = pltpu.async_copy(h_hbm.at[src_v.at[pl.ds(b1, BS)]],
                                      hbuf1, sg1)
                g0.wait()
                compute_edge(b0, hbuf0, mbuf0)
                for _k in range(BS // LANES):
                    idx0[pl.ds(_k * LANES, LANES)] = dst_v[pl.ds(b0 + _k * LANES, LANES)]
                s0 = pltpu.async_copy(mbuf0, acc_sh.at[idx0], ss0, add=True)
                g1.wait()
                compute_edge(b1, hbuf1, mbuf1)
                for _k in range(BS // LANES):
                    idx1[pl.ds(_k * LANES, LANES)] = dst_v[pl.ds(b1 + _k * LANES, LANES)]
                s1 = pltpu.async_copy(mbuf1, acc_sh.at[idx1], ss1, add=True)
                s0.wait()
                s1.wait()
                return 0

            lax.fori_loop(0, EBLK // (2 * BS), pair_body, 0)
            return 0

        lax.fori_loop(0, EPT // EBLK, blk_body, 0)

        pltpu.sync_copy(qedge_hbm.at[pl.ds(qbase, QPT)], src_v)

        def glob_body(i, _):
            sl = pl.ds(i * LANES, LANES)
            src_v[sl] = src_v[sl] + c * EHP
            return 0

        lax.fori_loop(0, QPT // LANES, glob_body, 0)
        pltpu.sync_copy(s_hbm.at[src_v], aux_v)
        pltpu.sync_copy(normq_hbm.at[pl.ds(qbase, QPT)], scale_v)

        def scaleq_body(i, _):
            sl = pl.ds(i * LANES, LANES)
            keep = aux_v[sl] != 0.0
            scale_v[sl] = scale_v[sl] * jnp.where(keep, 1.0, 0.0)
            return 0

        lax.fori_loop(0, QPT // LANES, scaleq_body, 0)
        pltpu.sync_copy(srcq_hbm.at[pl.ds(qbase, QPT)], src_v)
        pltpu.sync_copy(dstq_hbm.at[pl.ds(qbase, QPT)], dst_v)
        pltpu.sync_copy(etq_hbm.at[pl.ds(qbase, QPT)], et_v)
        pltpu.sync_copy(relf_hbm, tbl_t)

        def compute_qual(b, hbuf, qwbuf, mbuf):
            for k in range(BS // LANES):
                jk = j16 + k * LANES
                sc16 = scale_v[pl.ds(b + k * LANES, LANES)]
                et16 = et_v[pl.ds(b + k * LANES, LANES)]

                def f_body(f, _):
                    f16 = jnp.full((LANES,), f, jnp.int32)
                    hv = plsc.load_gather(hbuf, [jk, f16])
                    rv = plsc.load_gather(tbl_t, [et16, f16])
                    wv = plsc.load_gather(qwbuf, [jk, f16])
                    plsc.store_scatter(mbuf, [jk, f16], hv * rv * wv * sc16)
                    return 0

                lax.fori_loop(0, cw, f_body, 0, unroll=8)

        def qpair_body(p, _):
            b0 = p * 2 * BS
            b1 = b0 + BS
            g0 = pltpu.async_copy(h_hbm.at[src_v.at[pl.ds(b0, BS)]],
                                  hbuf0, sg0)
            q0 = pltpu.async_copy(qw_hbm.at[pl.ds(qbase + b0, BS)],
                                  qwbuf0, sq0)
            g1 = pltpu.async_copy(h_hbm.at[src_v.at[pl.ds(b1, BS)]],
                                  hbuf1, sg1)
            q1 = pltpu.async_copy(qw_hbm.at[pl.ds(qbase + b1, BS)],
                                  qwbuf1, sq1)
            g0.wait()
            q0.wait()
            compute_qual(b0, hbuf0, qwbuf0, mbuf0)
            for _k in range(BS // LANES):
                    idx0[pl.ds(_k * LANES, LANES)] = dst_v[pl.ds(b0 + _k * LANES, LANES)]
            s0 = pltpu.async_copy(mbuf0, acc_sh.at[idx0], ss0, add=True)
            g1.wait()
            q1.wait()
            compute_qual(b1, hbuf1, qwbuf1, mbuf1)
            for _k in range(BS // LANES):
                    idx1[pl.ds(_k * LANES, LANES)] = dst_v[pl.ds(b1 + _k * LANES, LANES)]
            s1 = pltpu.async_copy(mbuf1, acc_sh.at[idx1], ss1, add=True)
            s0.wait()
            s1.wait()
            return 0

        lax.fori_loop(0, QPT // (2 * BS), qpair_body, 0)
        plsc.subcore_barrier()
        pltpu.sync_copy(acc_sh.at[pl.ds(arow0, ART)],
                        a_out.at[pl.ds(c * NR + arow0, ART)])

    return _accum


_SC_ACCUM_A = _make_sc_accum(CW_A)
_SC_ACCUM_B = _make_sc_accum(CW_B)


def kernel(x, edge_index, edge_type, qualifier_index, rel_embs, fr_W, fr_b,
           w_in_0, w_out_0, w_loop_0, w_rel_0, w_q_0, loop_rel_0,
           w_in_1, w_out_1, w_loop_1, w_rel_1, w_q_1, loop_rel_1):
    pad_node = DUMMY + (jnp.arange(EHP - EH, dtype=jnp.int32) % 8)
    pad2 = jnp.broadcast_to(pad_node, (2, EHP - EH))

    def pad_edges(a, pad):
        return jnp.concatenate([a.reshape(2, EH), pad], axis=1).reshape(-1)

    src_p = pad_edges(edge_index[0], pad2)
    dst_p = pad_edges(edge_index[1], pad2)
    et_p = pad_edges(edge_type, jnp.zeros((2, EHP - EH), jnp.int32))

    pad_qn = DUMMY + (jnp.arange(NQHP - NQH, dtype=jnp.int32) % 8)
    pad_q2 = jnp.broadcast_to(pad_qn, (2, NQHP - NQH))
    pad_qe = EH + (jnp.arange(NQHP - NQH, dtype=jnp.int32) % 8)
    pad_qe2 = jnp.broadcast_to(pad_qe, (2, NQHP - NQH))

    def pad_quals(a, pad):
        return jnp.concatenate([a.reshape(2, NQH), pad], axis=1).reshape(-1)

    qrel_p = pad_quals(qualifier_index[0], jnp.zeros((2, NQHP - NQH), jnp.int32))
    qent_p = pad_quals(qualifier_index[1], pad_q2)
    qedge_p = pad_quals(qualifier_index[2], pad_qe2)

    norm, src_q, dst_q, et_q, norm_q = _sc_prep(src_p, dst_p, et_p, qedge_p)

    x_p = jnp.pad(x, ((0, NR - N), (0, 0)))
    h = _tc_matmul_bias(x_p, _pad_d(fr_W, (1,)), _pad_d(fr_b, (0,)), bm=1256)

    rel = _pad_d(rel_embs, (1,))
    layers = [
        (w_in_0, w_out_0, w_loop_0, w_rel_0, w_q_0, loop_rel_0),
        (w_in_1, w_out_1, w_loop_1, w_rel_1, w_q_1, loop_rel_1),
    ]
    for w_in, w_out, w_loop, w_rel, w_q, loop_rel in layers:
        w_in, w_out, w_loop, w_rel, w_q = (
            _pad_d(w, (0, 1)) for w in (w_in, w_out, w_loop, w_rel, w_q))
        loop_rel = _pad_d(loop_rel, (1,))
        rel_full = jnp.zeros((DP, DP), jnp.float32)
        rel_full = rel_full.at[:R].set(rel).at[R].set(loop_rel[0])
        relq, rel_next = _tc_relprep(rel_full, w_q, w_rel)

        q_emb, s, scale = _sc_qualprep(h, rel_full, qent_p, qrel_p, qedge_p,
                                       norm)
        qw = _tc_qw(q_emb, w_q, bm=1280)

        args = (dst_p, et_p, src_p, scale, s, src_q, dst_q, et_q, norm_q,
                qedge_p)
        acc_a = _SC_ACCUM_A(h[:, :CW_A], relq[:, :CW_A], rel_full[:, :CW_A],
                            *args, qw[:, :CW_A])
        acc_b = _SC_ACCUM_B(h[:, CW_A:], relq[:, CW_A:], rel_full[:, CW_A:],
                            *args, qw[:, CW_A:])

        h = _tc_combine(acc_a[:NR], acc_b[:NR], acc_a[NR:], acc_b[NR:], h,
                        w_in, w_out, w_loop, loop_rel, bm=1256)
        rel = rel_next[:R]

    return h[:N, :D], rel[:R, :D]


# speedup vs baseline: 2.4139x; 1.8425x over previous
import functools

import jax
import jax.numpy as jnp
from jax import lax
from jax.experimental import pallas as pl
from jax.experimental.pallas import tpu as pltpu
from jax.experimental.pallas import tpu_sc as plsc

N = 10000
FEAT = 1024
D = 200
DP = 208
R = 200
E = 160000
EH = E // 2
NQH = 40000 // 2

NS = 16
LANES = 16
BS = 32

NR = 10048
DUMMY = 10016
EHP = 81920
EPT = EHP // NS
EBLK = 1280
NQHP = 20480
QPT = NQHP // NS
ART = NR // NS
CW_A = 112
CW_B = DP - CW_A

_MESH = plsc.VectorSubcoreMesh(core_axis_name="c", subcore_axis_name="s")
_SC_PARAMS = pltpu.CompilerParams(use_tc_tiling_on_sc=False,
                                  needs_layout_passes=False)


def _pad_d(a, axes):
    pads = [(0, 0)] * a.ndim
    for ax in axes:
        pads[ax] = (0, DP - a.shape[ax])
    return jnp.pad(a, pads)


def _mm_bias_body(x_ref, w_ref, b_ref, o_ref):
    o_ref[...] = jnp.dot(x_ref[...], w_ref[...],
                         preferred_element_type=jnp.float32) + b_ref[...]


def _tc_matmul_bias(x, w, b, bm):
    m, k = x.shape
    n = w.shape[1]
    return pl.pallas_call(
        _mm_bias_body,
        grid=(m // bm,),
        in_specs=[
            pl.BlockSpec((bm, k), lambda i: (i, 0)),
            pl.BlockSpec((k, n), lambda i: (0, 0)),
            pl.BlockSpec((1, n), lambda i: (0, 0)),
        ],
        out_specs=pl.BlockSpec((bm, n), lambda i: (i, 0)),
        out_shape=jax.ShapeDtypeStruct((m, n), jnp.float32),
    )(x, w, b.reshape(1, n))


def _qw_body(q_ref, w_ref, o_ref):
    o_ref[...] = jnp.dot(q_ref[...], w_ref[...],
                         preferred_element_type=jnp.float32)


def _tc_qw(q_emb, w_q, bm):
    m = q_emb.shape[0]
    return pl.pallas_call(
        _qw_body,
        grid=(m // bm,),
        in_specs=[
            pl.BlockSpec((bm, DP), lambda i: (i, 0)),
            pl.BlockSpec((DP, DP), lambda i: (0, 0)),
        ],
        out_specs=pl.BlockSpec((bm, DP), lambda i: (i, 0)),
        out_shape=jax.ShapeDtypeStruct((m, DP), jnp.float32),
    )(q_emb, w_q)


def _relprep_body(rf_ref, wq_ref, wr_ref, relq_ref, reln_ref):
    wq = wq_ref[...]
    cq = jnp.sum(wq, axis=0, keepdims=True)
    relq_ref[...] = rf_ref[...] * cq
    reln_ref[...] = jnp.dot(rf_ref[...], wr_ref[...],
                            preferred_element_type=jnp.float32)


def _tc_relprep(rel_full, w_q, w_rel):
    return pl.pallas_call(
        _relprep_body,
        in_specs=[pl.BlockSpec((DP, DP), lambda: (0, 0))] * 3,
        out_specs=[pl.BlockSpec((DP, DP), lambda: (0, 0))] * 2,
        out_shape=[jax.ShapeDtypeStruct((DP, DP), jnp.float32)] * 2,
        grid=(),
    )(rel_full, w_q, w_rel)


def _combine_body(aina_ref, ainb_ref, aouta_ref, aoutb_ref, h_ref,
                  wi_ref, wo_ref, wl_ref, lr_ref, o_ref):
    f32 = jnp.float32
    acc = jnp.dot(aina_ref[...], wi_ref[:CW_A, :], preferred_element_type=f32)
    acc += jnp.dot(ainb_ref[...], wi_ref[CW_A:, :], preferred_element_type=f32)
    acc += jnp.dot(aouta_ref[...], wo_ref[:CW_A, :], preferred_element_type=f32)
    acc += jnp.dot(aoutb_ref[...], wo_ref[CW_A:, :], preferred_element_type=f32)
    acc += jnp.dot(h_ref[...] * lr_ref[...], wl_ref[...],
                   preferred_element_type=f32)
    o_ref[...] = jnp.tanh(acc * (1.0 / 3.0))


def _tc_combine(aina, ainb, aouta, aoutb, h, w_in, w_out, w_loop, loop_rel,
                bm):
    m = h.shape[0]
    return pl.pallas_call(
        _combine_body,
        grid=(m // bm,),
        in_specs=[
            pl.BlockSpec((bm, CW_A), lambda i: (i, 0)),
            pl.BlockSpec((bm, CW_B), lambda i: (i, 0)),
            pl.BlockSpec((bm, CW_A), lambda i: (i, 0)),
            pl.BlockSpec((bm, CW_B), lambda i: (i, 0)),
            pl.BlockSpec((bm, DP), lambda i: (i, 0)),
            pl.BlockSpec((DP, DP), lambda i: (0, 0)),
            pl.BlockSpec((DP, DP), lambda i: (0, 0)),
            pl.BlockSpec((DP, DP), lambda i: (0, 0)),
            pl.BlockSpec((1, DP), lambda i: (0, 0)),
        ],
        out_specs=pl.BlockSpec((bm, DP), lambda i: (i, 0)),
        out_shape=jax.ShapeDtypeStruct((m, DP), jnp.float32),
    )(aina, ainb, aouta, aoutb, h, w_in, w_out, w_loop, loop_rel)


def _rsqrt16(d):
    i = plsc.bitcast(d, jnp.int32)
    i = jnp.int32(0x5F3759DF) - lax.shift_right_logical(i, 1)
    y = plsc.bitcast(i, jnp.float32)
    for _ in range(3):
        y = y * (1.5 - 0.5 * d * y * y)
    return jnp.where(d > 0, y, 0.0)


def _fill(ref, n, value):
    v = jnp.full((LANES,), value, ref.dtype)

    def body(i, _):
        ref[pl.ds(i * LANES, LANES)] = v
        return 0

    lax.fori_loop(0, n // LANES, body, 0)


@functools.partial(
    pl.kernel,
    out_type=[
        jax.ShapeDtypeStruct((2 * EHP,), jnp.float32),
        jax.ShapeDtypeStruct((2 * NQHP,), jnp.int32),
        jax.ShapeDtypeStruct((2 * NQHP,), jnp.int32),
        jax.ShapeDtypeStruct((2 * NQHP,), jnp.int32),
        jax.ShapeDtypeStruct((2 * NQHP,), jnp.float32),
    ],
    mesh=_MESH,
    compiler_params=_SC_PARAMS,
    scratch_types=[
        pltpu.VMEM_SHARED((NR,), jnp.float32),
        pltpu.VMEM((EPT,), jnp.int32),
        pltpu.VMEM((EPT,), jnp.int32),
        pltpu.VMEM((EPT,), jnp.float32),
        pltpu.VMEM((640,), jnp.float32),
        pltpu.VMEM((NR,), jnp.float32),
        pltpu.VMEM((EHP,), jnp.int32),
        pltpu.VMEM((QPT,), jnp.int32),
        pltpu.VMEM((QPT,), jnp.int32),
        pltpu.VMEM((QPT,), jnp.int32),
        pltpu.VMEM((QPT,), jnp.float32),
    ],
)
def _sc_prep(src_hbm, dst_hbm, et_hbm, qedge_hbm,
             norm_out, srcq_out, dstq_out, etq_out, normq_out,
             deg_sh, src_v, dst_v, val_v, zb_v, dinv_v, tbl_v,
             qe_v, sq_v, dq_v, nq_v):
    c = lax.axis_index("c")
    t = lax.axis_index("s")
    ebase = c * EHP + t * EPT
    qbase = c * NQHP + t * QPT

    pltpu.sync_copy(src_hbm.at[pl.ds(ebase, EPT)], src_v)
    pltpu.sync_copy(dst_hbm.at[pl.ds(ebase, EPT)], dst_v)

    _fill(zb_v, 640, 0.0)
    zbase = jnp.minimum(t * 640, NR - 640)
    pltpu.sync_copy(zb_v, deg_sh.at[pl.ds(zbase, 640)])
    _fill(val_v, EPT, 1.0)
    plsc.subcore_barrier()

    pltpu.sync_copy(val_v, deg_sh.at[src_v], add=True)
    plsc.subcore_barrier()

    pltpu.sync_copy(deg_sh, dinv_v)

    def inv_body(i, _):
        d = dinv_v[pl.ds(i * LANES, LANES)]
        dinv_v[pl.ds(i * LANES, LANES)] = _rsqrt16(d)
        return 0

    lax.fori_loop(0, NR // LANES, inv_body, 0)

    def norm_body(i, _):
        sl = pl.ds(i * LANES, LANES)
        a = plsc.load_gather(dinv_v, [src_v[sl]])
        b = plsc.load_gather(dinv_v, [dst_v[sl]])
        val_v[sl] = a * b
        return 0

    lax.fori_loop(0, EPT // LANES, norm_body, 0)
    pltpu.sync_copy(val_v, norm_out.at[pl.ds(ebase, EPT)])

    pltpu.sync_copy(qedge_hbm.at[pl.ds(qbase, QPT)], qe_v)

    def gather_q(table_hbm, out_v):
        pltpu.sync_copy(table_hbm.at[pl.ds(c * EHP, EHP)], tbl_v)

        def body(i, _):
            sl = pl.ds(i * LANES, LANES)
            out_v[sl] = plsc.load_gather(tbl_v, [qe_v[sl]])
            return 0

        lax.fori_loop(0, QPT // LANES, body, 0)

    gather_q(src_hbm, sq_v)
    pltpu.sync_copy(sq_v, srcq_out.at[pl.ds(qbase, QPT)])
    gather_q(dst_hbm, dq_v)
    pltpu.sync_copy(dq_v, dstq_out.at[pl.ds(qbase, QPT)])

    def normq_body(i, _):
        sl = pl.ds(i * LANES, LANES)
        a = plsc.load_gather(dinv_v, [sq_v[sl]])
        b = plsc.load_gather(dinv_v, [dq_v[sl]])
        nq_v[sl] = a * b
        return 0

    lax.fori_loop(0, QPT // LANES, normq_body, 0)
    pltpu.sync_copy(nq_v, normq_out.at[pl.ds(qbase, QPT)])

    gather_q(et_hbm, dq_v)
    pltpu.sync_copy(dq_v, etq_out.at[pl.ds(qbase, QPT)])


@functools.partial(
    pl.kernel,
    out_type=[
        jax.ShapeDtypeStruct((2 * NQHP, DP), jnp.float32),
        jax.ShapeDtypeStruct((2 * EHP,), jnp.float32),
        jax.ShapeDtypeStruct((2 * EHP,), jnp.float32),
    ],
    mesh=_MESH,
    compiler_params=_SC_PARAMS,
    scratch_types=[
        pltpu.VMEM_SHARED((EHP,), jnp.float32),
        pltpu.VMEM((QPT,), jnp.int32),
        pltpu.VMEM((QPT,), jnp.int32),
        pltpu.VMEM((QPT,), jnp.int32),
        pltpu.VMEM((QPT,), jnp.float32),
        pltpu.VMEM((DP, DP), jnp.float32),
        pltpu.VMEM((BS, DP), jnp.float32),
        pltpu.VMEM((BS, DP), jnp.float32),
        pltpu.VMEM((BS, DP), jnp.float32),
        pltpu.VMEM((BS, DP), jnp.float32),
        pltpu.VMEM((EPT,), jnp.float32),
        pltpu.VMEM((EPT,), jnp.float32),
        pltpu.VMEM((640,), jnp.float32),
        pltpu.SemaphoreType.DMA,
        pltpu.SemaphoreType.DMA,
        pltpu.SemaphoreType.DMA,
        pltpu.SemaphoreType.DMA,
    ],
)
def _sc_qualprep(h_hbm, relf_hbm, qent_hbm, qrel_hbm, qedge_hbm, norm_hbm,
                 qemb_out, s_out, scale_out,
                 s_sh, qent_v, qrel_v, qe_v, r_v, relf_t,
                 hbuf0, hbuf1, qbuf0, qbuf1, nrm_v, sl_v, zb_v,
                 sg0, sg1, sw0, sw1):
    c = lax.axis_index("c")
    t = lax.axis_index("s")
    qbase = c * NQHP + t * QPT
    ebase = c * EHP + t * EPT

    pltpu.sync_copy(qent_hbm.at[pl.ds(qbase, QPT)], qent_v)
    pltpu.sync_copy(qrel_hbm.at[pl.ds(qbase, QPT)], qrel_v)
    pltpu.sync_copy(qedge_hbm.at[pl.ds(qbase, QPT)], qe_v)
    pltpu.sync_copy(relf_hbm, relf_t)

    _fill(zb_v, 640, 0.0)
    for i in range(EPT // 640):
        pltpu.sync_copy(zb_v, s_sh.at[pl.ds(t * EPT + i * 640, 640)])
    plsc.subcore_barrier()

    j16 = lax.iota(jnp.int32, LANES)

    def compute(qb, hbuf, qbuf):
        for k in range(BS // LANES):
            qr16 = qrel_v[pl.ds(qb + k * LANES, LANES)]
            rsum = jnp.zeros((LANES,), jnp.float32)
            for jj in range(LANES):
                j = k * LANES + jj
                qr_j = jnp.full((LANES,), qr16[jj], jnp.int32)
                racc = jnp.zeros((LANES,), jnp.float32)
                for kk in range(DP // LANES):
                    fs = pl.ds(kk * LANES, LANES)
                    rv = plsc.load_gather(relf_t, [qr_j, j16 + kk * LANES])
                    q = hbuf[j, fs] * rv
                    qbuf[j, fs] = q
                    racc = racc + q
                rsum = rsum + jnp.where(j16 == jj, jnp.sum(racc), 0.0)
            r_v[pl.ds(qb + k * LANES, LANES)] = rsum

    def pair_body(p, _):
        b0 = p * 2 * BS
        b1 = b0 + BS
        g0 = pltpu.async_copy(h_hbm.at[qent_v.at[pl.ds(b0, BS)]], hbuf0, sg0)
        g1 = pltpu.async_copy(h_hbm.at[qent_v.at[pl.ds(b1, BS)]], hbuf1, sg1)
        g0.wait()
        compute(b0, hbuf0, qbuf0)
        w0 = pltpu.async_copy(qbuf0, qemb_out.at[pl.ds(qbase + b0, BS)], sw0)
        g1.wait()
        compute(b1, hbuf1, qbuf1)
        w1 = pltpu.async_copy(qbuf1, qemb_out.at[pl.ds(qbase + b1, BS)], sw1)
        w0.wait()
        w1.wait()
        return 0

    lax.fori_loop(0, QPT // (2 * BS), pair_body, 0)

    pltpu.sync_copy(r_v, s_sh.at[qe_v], add=True)
    plsc.subcore_barrier()
    pltpu.sync_copy(s_sh.at[pl.ds(t * EPT, EPT)], sl_v)
    pltpu.sync_copy(sl_v, s_out.at[pl.ds(ebase, EPT)])
    pltpu.sync_copy(norm_hbm.at[pl.ds(ebase, EPT)], nrm_v)

    def scale_body(i, _):
        sl = pl.ds(i * LANES, LANES)
        keep = sl_v[sl] == 0.0
        nrm_v[sl] = nrm_v[sl] * jnp.where(keep, 1.0, 0.0)
        return 0

    lax.fori_loop(0, EPT // LANES, scale_body, 0)
    pltpu.sync_copy(nrm_v, scale_out.at[pl.ds(ebase, EPT)])


def _make_sc_accum(cw):
    @functools.partial(
        pl.kernel,
        out_type=jax.ShapeDtypeStruct((2 * NR, cw), jnp.float32),
        mesh=_MESH,
        compiler_params=_SC_PARAMS,
        scratch_types=[
            pltpu.VMEM_SHARED((NR, cw), jnp.float32),
            pltpu.VMEM((DP, cw), jnp.float32),
            pltpu.VMEM((EBLK,), jnp.int32),
            pltpu.VMEM((EBLK,), jnp.int32),
            pltpu.VMEM((EBLK,), jnp.int32),
            pltpu.VMEM((EBLK,), jnp.float32),
            pltpu.VMEM((BS, cw), jnp.float32),
            pltpu.VMEM((BS, cw), jnp.float32),
            pltpu.VMEM((BS, cw), jnp.float32),
            pltpu.VMEM((BS, cw), jnp.float32),
            pltpu.VMEM((BS, cw), jnp.float32),
            pltpu.VMEM((BS, cw), jnp.float32),
            pltpu.VMEM((BS,), jnp.int32),
            pltpu.VMEM((BS,), jnp.int32),
            pltpu.VMEM((4, cw), jnp.float32),
            pltpu.VMEM((EBLK,), jnp.float32),
            pltpu.SemaphoreType.DMA,
            pltpu.SemaphoreType.DMA,
            pltpu.SemaphoreType.DMA,
            pltpu.SemaphoreType.DMA,
            pltpu.SemaphoreType.DMA,
            pltpu.SemaphoreType.DMA,
        ],
    )
    def _accum(h_hbm, relq_hbm, relf_hbm, dst_hbm, et_hbm, src_hbm,
               scale_hbm, s_hbm, srcq_hbm, dstq_hbm, etq_hbm, normq_hbm,
               qedge_hbm, qw_hbm,
               a_out,
               acc_sh, tbl_t, src_v, dst_v, et_v, scale_v,
               hbuf0, hbuf1, mbuf0, mbuf1, qwbuf0, qwbuf1,
               idx0, idx1, zrows, aux_v,
               sg0, sg1, ss0, ss1, sq0, sq1):
        c = lax.axis_index("c")
        t = lax.axis_index("s")
        ebase = c * EHP + t * EPT
        qbase = c * NQHP + t * QPT
        arow0 = t * ART

        for i in range(4):
            for k in range(cw // LANES):
                zrows[i, pl.ds(k * LANES, LANES)] = jnp.zeros((LANES,),
                                                              jnp.float32)

        def zero_body(i, _):
            pltpu.sync_copy(zrows, acc_sh.at[pl.ds(arow0 + i * 4, 4)])
            return 0

        lax.fori_loop(0, ART // 4, zero_body, 0)
        pltpu.sync_copy(relq_hbm, tbl_t)
        plsc.subcore_barrier()

        j16 = lax.iota(jnp.int32, LANES)

        def compute_edge(b, hbuf, mbuf):
            for k in range(BS // LANES):
                et16 = et_v[pl.ds(b + k * LANES, LANES)]
                sc16 = scale_v[pl.ds(b + k * LANES, LANES)]
                for jj in range(LANES):
                    j = k * LANES + jj
                    et_j = jnp.full((LANES,), et16[jj], jnp.int32)
                    scv = jnp.full((LANES,), sc16[jj], jnp.float32)
                    for kk in range(cw // LANES):
                        fs = pl.ds(kk * LANES, LANES)
                        rv = plsc.load_gather(tbl_t, [et_j, j16 + kk * LANES])
                        mbuf[j, fs] = hbuf[j, fs] * rv * scv

        def blk_body(blk, _):
            bbase = ebase + blk * EBLK
            pltpu.sync_copy(src_hbm.at[pl.ds(bbase, EBLK)], src_v)
            pltpu.sync_copy(dst_hbm.at[pl.ds(bbase, EBLK)], dst_v)
            pltpu.sync_copy(et_hbm.at[pl.ds(bbase, EBLK)], et_v)
            pltpu.sync_copy(scale_hbm.at[pl.ds(bbase, EBLK)], scale_v)

            def pair_body(p, _):
                b0 = p * 2 * BS
                b1 = b0 + BS
                g0 = pltpu.async_copy(h_hbm.at[src_v.at[pl.ds(b0, BS)]],
                                      hbuf0, sg0)
                g1 = pltpu.async_copy(h_hbm.at[src_v.at[pl.ds(b1, BS)]],
                                      hbuf1, sg1)
                g0.wait()
                compute_edge(b0, hbuf0, mbuf0)
                for _k in range(BS // LANES):
                    idx0[pl.ds(_k * LANES, LANES)] = dst_v[pl.ds(b0 + _k * LANES, LANES)]
                s0 = pltpu.async_copy(mbuf0, acc_sh.at[idx0], ss0, add=True)
                g1.wait()
                compute_edge(b1, hbuf1, mbuf1)
                for _k in range(BS // LANES):
                    idx1[pl.ds(_k * LANES, LANES)] = dst_v[pl.ds(b1 + _k * LANES, LANES)]
                s1 = pltpu.async_copy(mbuf1, acc_sh.at[idx1], ss1, add=True)
                s0.wait()
                s1.wait()
                return 0

            lax.fori_loop(0, EBLK // (2 * BS), pair_body, 0)
            return 0

        lax.fori_loop(0, EPT // EBLK, blk_body, 0)

        pltpu.sync_copy(qedge_hbm.at[pl.ds(qbase, QPT)], src_v)

        def glob_body(i, _):
            sl = pl.ds(i * LANES, LANES)
            src_v[sl] = src_v[sl] + c * EHP
            return 0

        lax.fori_loop(0, QPT // LANES, glob_body, 0)
        pltpu.sync_copy(s_hbm.at[src_v], aux_v)
        pltpu.sync_copy(normq_hbm.at[pl.ds(qbase, QPT)], scale_v)

        def scaleq_body(i, _):
            sl = pl.ds(i * LANES, LANES)
            keep = aux_v[sl] != 0.0
            scale_v[sl] = scale_v[sl] * jnp.where(keep, 1.0, 0.0)
            return 0

        lax.fori_loop(0, QPT // LANES, scaleq_body, 0)
        pltpu.sync_copy(srcq_hbm.at[pl.ds(qbase, QPT)], src_v)
        pltpu.sync_copy(dstq_hbm.at[pl.ds(qbase, QPT)], dst_v)
        pltpu.sync_copy(etq_hbm.at[pl.ds(qbase, QPT)], et_v)
        pltpu.sync_copy(relf_hbm, tbl_t)

        def compute_qual(b, hbuf, qwbuf, mbuf):
            for k in range(BS // LANES):
                et16 = et_v[pl.ds(b + k * LANES, LANES)]
                sc16 = scale_v[pl.ds(b + k * LANES, LANES)]
                for jj in range(LANES):
                    j = k * LANES + jj
                    et_j = jnp.full((LANES,), et16[jj], jnp.int32)
                    scv = jnp.full((LANES,), sc16[jj], jnp.float32)
                    for kk in range(cw // LANES):
                        fs = pl.ds(kk * LANES, LANES)
                        rv = plsc.load_gather(tbl_t, [et_j, j16 + kk * LANES])
                        mbuf[j, fs] = hbuf[j, fs] * rv * qwbuf[j, fs] * scv

        def qpair_body(p, _):
            b0 = p * 2 * BS
            b1 = b0 + BS
            g0 = pltpu.async_copy(h_hbm.at[src_v.at[pl.ds(b0, BS)]],
                                  hbuf0, sg0)
            q0 = pltpu.async_copy(qw_hbm.at[pl.ds(qbase + b0, BS)],
                                  qwbuf0, sq0)
            g1 = pltpu.async_copy(h_hbm.at[src_v.at[pl.ds(b1, BS)]],
                                  hbuf1, sg1)
            q1 = pltpu.async_copy(qw_hbm.at[pl.ds(qbase + b1, BS)],
                                  qwbuf1, sq1)
            g0.wait()
            q0.wait()
            compute_qual(b0, hbuf0, qwbuf0, mbuf0)
            for _k in range(BS // LANES):
                    idx0[pl.ds(_k * LANES, LANES)] = dst_v[pl.ds(b0 + _k * LANES, LANES)]
            s0 = pltpu.async_copy(mbuf0, acc_sh.at[idx0], ss0, add=True)
            g1.wait()
            q1.wait()
            compute_qual(b1, hbuf1, qwbuf1, mbuf1)
            for _k in range(BS // LANES):
                    idx1[pl.ds(_k * LANES, LANES)] = dst_v[pl.ds(b1 + _k * LANES, LANES)]
            s1 = pltpu.async_copy(mbuf1, acc_sh.at[idx1], ss1, add=True)
            s0.wait()
            s1.wait()
            return 0

        lax.fori_loop(0, QPT // (2 * BS), qpair_body, 0)
        plsc.subcore_barrier()
        pltpu.sync_copy(acc_sh.at[pl.ds(arow0, ART)],
                        a_out.at[pl.ds(c * NR + arow0, ART)])

    return _accum


_SC_ACCUM_A = _make_sc_accum(CW_A)
_SC_ACCUM_B = _make_sc_accum(CW_B)


def kernel(x, edge_index, edge_type, qualifier_index, rel_embs, fr_W, fr_b,
           w_in_0, w_out_0, w_loop_0, w_rel_0, w_q_0, loop_rel_0,
           w_in_1, w_out_1, w_loop_1, w_rel_1, w_q_1, loop_rel_1):
    pad_node = DUMMY + (jnp.arange(EHP - EH, dtype=jnp.int32) % 8)
    pad2 = jnp.broadcast_to(pad_node, (2, EHP - EH))

    def pad_edges(a, pad):
        return jnp.concatenate([a.reshape(2, EH), pad], axis=1).reshape(-1)

    src_p = pad_edges(edge_index[0], pad2)
    dst_p = pad_edges(edge_index[1], pad2)
    et_p = pad_edges(edge_type, jnp.zeros((2, EHP - EH), jnp.int32))

    pad_qn = DUMMY + (jnp.arange(NQHP - NQH, dtype=jnp.int32) % 8)
    pad_q2 = jnp.broadcast_to(pad_qn, (2, NQHP - NQH))
    pad_qe = EH + (jnp.arange(NQHP - NQH, dtype=jnp.int32) % 8)
    pad_qe2 = jnp.broadcast_to(pad_qe, (2, NQHP - NQH))

    def pad_quals(a, pad):
        return jnp.concatenate([a.reshape(2, NQH), pad], axis=1).reshape(-1)

    qrel_p = pad_quals(qualifier_index[0], jnp.zeros((2, NQHP - NQH), jnp.int32))
    qent_p = pad_quals(qualifier_index[1], pad_q2)
    qedge_p = pad_quals(qualifier_index[2], pad_qe2)

    norm, src_q, dst_q, et_q, norm_q = _sc_prep(src_p, dst_p, et_p, qedge_p)

    x_p = jnp.pad(x, ((0, NR - N), (0, 0)))
    h = _tc_matmul_bias(x_p, _pad_d(fr_W, (1,)), _pad_d(fr_b, (0,)), bm=1256)

    rel = _pad_d(rel_embs, (1,))
    layers = [
        (w_in_0, w_out_0, w_loop_0, w_rel_0, w_q_0, loop_rel_0),
        (w_in_1, w_out_1, w_loop_1, w_rel_1, w_q_1, loop_rel_1),
    ]
    for w_in, w_out, w_loop, w_rel, w_q, loop_rel in layers:
        w_in, w_out, w_loop, w_rel, w_q = (
            _pad_d(w, (0, 1)) for w in (w_in, w_out, w_loop, w_rel, w_q))
        loop_rel = _pad_d(loop_rel, (1,))
        rel_full = jnp.zeros((DP, DP), jnp.float32)
        rel_full = rel_full.at[:R].set(rel).at[R].set(loop_rel[0])
        relq, rel_next = _tc_relprep(rel_full, w_q, w_rel)

        q_emb, s, scale = _sc_qualprep(h, rel_full, qent_p, qrel_p, qedge_p,
                                       norm)
        qw = _tc_qw(q_emb, w_q, bm=1280)

        args = (dst_p, et_p, src_p, scale, s, src_q, dst_q, et_q, norm_q,
                qedge_p)
        acc_a = _SC_ACCUM_A(h[:, :CW_A], relq[:, :CW_A], rel_full[:, :CW_A],
                            *args, qw[:, :CW_A])
        acc_b = _SC_ACCUM_B(h[:, CW_A:], relq[:, CW_A:], rel_full[:, CW_A:],
                            *args, qw[:, CW_A:])

        h = _tc_combine(acc_a[:NR], acc_b[:NR], acc_a[NR:], acc_b[NR:], h,
                        w_in, w_out, w_loop, loop_rel, bm=1256)
        rel = rel_next[:R]

    return h[:N, :D], rel[:R, :D]


# speedup vs baseline: 2.7954x; 1.1580x over previous
import functools

import jax
import jax.numpy as jnp
from jax import lax
from jax.experimental import pallas as pl
from jax.experimental.pallas import tpu as pltpu
from jax.experimental.pallas import tpu_sc as plsc

N = 10000
FEAT = 1024
D = 200
DP = 208
R = 200
E = 160000
EH = E // 2
NQH = 40000 // 2

NS = 16
LANES = 16
BS = 16

NR = 10048
DUMMY = 10016
EHP = 81920
EPT = EHP // NS
EBLK = 1280
NQHP = 20480
QPT = NQHP // NS
ART = NR // NS
CW_A = 112
CW_B = DP - CW_A

_MESH = plsc.VectorSubcoreMesh(core_axis_name="c", subcore_axis_name="s")
_SC_PARAMS = pltpu.CompilerParams(use_tc_tiling_on_sc=False,
                                  needs_layout_passes=False)


def _pad_d(a, axes):
    pads = [(0, 0)] * a.ndim
    for ax in axes:
        pads[ax] = (0, DP - a.shape[ax])
    return jnp.pad(a, pads)


def _mm_bias_body(x_ref, w_ref, b_ref, o_ref):
    o_ref[...] = jnp.dot(x_ref[...], w_ref[...],
                         preferred_element_type=jnp.float32) + b_ref[...]


def _tc_matmul_bias(x, w, b, bm):
    m, k = x.shape
    n = w.shape[1]
    return pl.pallas_call(
        _mm_bias_body,
        grid=(m // bm,),
        in_specs=[
            pl.BlockSpec((bm, k), lambda i: (i, 0)),
            pl.BlockSpec((k, n), lambda i: (0, 0)),
            pl.BlockSpec((1, n), lambda i: (0, 0)),
        ],
        out_specs=pl.BlockSpec((bm, n), lambda i: (i, 0)),
        out_shape=jax.ShapeDtypeStruct((m, n), jnp.float32),
    )(x, w, b.reshape(1, n))


def _qw_body(q_ref, w_ref, o_ref):
    o_ref[...] = jnp.dot(q_ref[...], w_ref[...],
                         preferred_element_type=jnp.float32)


def _tc_qw(q_emb, w_q, bm):
    m = q_emb.shape[0]
    return pl.pallas_call(
        _qw_body,
        grid=(m // bm,),
        in_specs=[
            pl.BlockSpec((bm, DP), lambda i: (i, 0)),
            pl.BlockSpec((DP, DP), lambda i: (0, 0)),
        ],
        out_specs=pl.BlockSpec((bm, DP), lambda i: (i, 0)),
        out_shape=jax.ShapeDtypeStruct((m, DP), jnp.float32),
    )(q_emb, w_q)


def _relprep_body(rf_ref, wq_ref, wr_ref, relq_ref, reln_ref):
    wq = wq_ref[...]
    cq = jnp.sum(wq, axis=0, keepdims=True)
    relq_ref[...] = rf_ref[...] * cq
    reln_ref[...] = jnp.dot(rf_ref[...], wr_ref[...],
                            preferred_element_type=jnp.float32)


def _tc_relprep(rel_full, w_q, w_rel):
    return pl.pallas_call(
        _relprep_body,
        in_specs=[pl.BlockSpec((DP, DP), lambda: (0, 0))] * 3,
        out_specs=[pl.BlockSpec((DP, DP), lambda: (0, 0))] * 2,
        out_shape=[jax.ShapeDtypeStruct((DP, DP), jnp.float32)] * 2,
        grid=(),
    )(rel_full, w_q, w_rel)


def _combine_body(aina_ref, ainb_ref, aouta_ref, aoutb_ref, h_ref,
                  wi_ref, wo_ref, wl_ref, lr_ref, o_ref):
    f32 = jnp.float32
    acc = jnp.dot(aina_ref[...], wi_ref[:CW_A, :], preferred_element_type=f32)
    acc += jnp.dot(ainb_ref[...], wi_ref[CW_A:, :], preferred_element_type=f32)
    acc += jnp.dot(aouta_ref[...], wo_ref[:CW_A, :], preferred_element_type=f32)
    acc += jnp.dot(aoutb_ref[...], wo_ref[CW_A:, :], preferred_element_type=f32)
    acc += jnp.dot(h_ref[...] * lr_ref[...], wl_ref[...],
                   preferred_element_type=f32)
    o_ref[...] = jnp.tanh(acc * (1.0 / 3.0))


def _tc_combine(aina, ainb, aouta, aoutb, h, w_in, w_out, w_loop, loop_rel,
                bm):
    m = h.shape[0]
    return pl.pallas_call(
        _combine_body,
        grid=(m // bm,),
        in_specs=[
            pl.BlockSpec((bm, CW_A), lambda i: (i, 0)),
            pl.BlockSpec((bm, CW_B), lambda i: (i, 0)),
            pl.BlockSpec((bm, CW_A), lambda i: (i, 0)),
            pl.BlockSpec((bm, CW_B), lambda i: (i, 0)),
            pl.BlockSpec((bm, DP), lambda i: (i, 0)),
            pl.BlockSpec((DP, DP), lambda i: (0, 0)),
            pl.BlockSpec((DP, DP), lambda i: (0, 0)),
            pl.BlockSpec((DP, DP), lambda i: (0, 0)),
            pl.BlockSpec((1, DP), lambda i: (0, 0)),
        ],
        out_specs=pl.BlockSpec((bm, DP), lambda i: (i, 0)),
        out_shape=jax.ShapeDtypeStruct((m, DP), jnp.float32),
    )(aina, ainb, aouta, aoutb, h, w_in, w_out, w_loop, loop_rel)


def _rsqrt16(d):
    i = plsc.bitcast(d, jnp.int32)
    i = jnp.int32(0x5F3759DF) - lax.shift_right_logical(i, 1)
    y = plsc.bitcast(i, jnp.float32)
    for _ in range(3):
        y = y * (1.5 - 0.5 * d * y * y)
    return jnp.where(d > 0, y, 0.0)


def _fill(ref, n, value):
    v = jnp.full((LANES,), value, ref.dtype)

    def body(i, _):
        ref[pl.ds(i * LANES, LANES)] = v
        return 0

    lax.fori_loop(0, n // LANES, body, 0)


@functools.partial(
    pl.kernel,
    out_type=[
        jax.ShapeDtypeStruct((2 * EHP,), jnp.float32),
        jax.ShapeDtypeStruct((2 * NQHP,), jnp.int32),
        jax.ShapeDtypeStruct((2 * NQHP,), jnp.int32),
        jax.ShapeDtypeStruct((2 * NQHP,), jnp.int32),
        jax.ShapeDtypeStruct((2 * NQHP,), jnp.float32),
    ],
    mesh=_MESH,
    compiler_params=_SC_PARAMS,
    scratch_types=[
        pltpu.VMEM_SHARED((NR,), jnp.float32),
        pltpu.VMEM((EPT,), jnp.int32),
        pltpu.VMEM((EPT,), jnp.int32),
        pltpu.VMEM((EPT,), jnp.float32),
        pltpu.VMEM((640,), jnp.float32),
        pltpu.VMEM((NR,), jnp.float32),
        pltpu.VMEM((EHP,), jnp.int32),
        pltpu.VMEM((QPT,), jnp.int32),
        pltpu.VMEM((QPT,), jnp.int32),
        pltpu.VMEM((QPT,), jnp.int32),
        pltpu.VMEM((QPT,), jnp.float32),
    ],
)
def _sc_prep(src_hbm, dst_hbm, et_hbm, qedge_hbm,
             norm_out, srcq_out, dstq_out, etq_out, normq_out,
             deg_sh, src_v, dst_v, val_v, zb_v, dinv_v, tbl_v,
             qe_v, sq_v, dq_v, nq_v):
    c = lax.axis_index("c")
    t = lax.axis_index("s")
    ebase = c * EHP + t * EPT
    qbase = c * NQHP + t * QPT

    pltpu.sync_copy(src_hbm.at[pl.ds(ebase, EPT)], src_v)
    pltpu.sync_copy(dst_hbm.at[pl.ds(ebase, EPT)], dst_v)

    _fill(zb_v, 640, 0.0)
    zbase = jnp.minimum(t * 640, NR - 640)
    pltpu.sync_copy(zb_v, deg_sh.at[pl.ds(zbase, 640)])
    _fill(val_v, EPT, 1.0)
    plsc.subcore_barrier()

    pltpu.sync_copy(val_v, deg_sh.at[src_v], add=True)
    plsc.subcore_barrier()

    pltpu.sync_copy(deg_sh, dinv_v)

    def inv_body(i, _):
        d = dinv_v[pl.ds(i * LANES, LANES)]
        dinv_v[pl.ds(i * LANES, LANES)] = _rsqrt16(d)
        return 0

    lax.fori_loop(0, NR // LANES, inv_body, 0)

    def norm_body(i, _):
        sl = pl.ds(i * LANES, LANES)
        a = plsc.load_gather(dinv_v, [src_v[sl]])
        b = plsc.load_gather(dinv_v, [dst_v[sl]])
        val_v[sl] = a * b
        return 0

    lax.fori_loop(0, EPT // LANES, norm_body, 0)
    pltpu.sync_copy(val_v, norm_out.at[pl.ds(ebase, EPT)])

    pltpu.sync_copy(qedge_hbm.at[pl.ds(qbase, QPT)], qe_v)

    def gather_q(table_hbm, out_v):
        pltpu.sync_copy(table_hbm.at[pl.ds(c * EHP, EHP)], tbl_v)

        def body(i, _):
            sl = pl.ds(i * LANES, LANES)
            out_v[sl] = plsc.load_gather(tbl_v, [qe_v[sl]])
            return 0

        lax.fori_loop(0, QPT // LANES, body, 0)

    gather_q(src_hbm, sq_v)
    pltpu.sync_copy(sq_v, srcq_out.at[pl.ds(qbase, QPT)])
    gather_q(dst_hbm, dq_v)
    pltpu.sync_copy(dq_v, dstq_out.at[pl.ds(qbase, QPT)])

    def normq_body(i, _):
        sl = pl.ds(i * LANES, LANES)
        a = plsc.load_gather(dinv_v, [sq_v[sl]])
        b = plsc.load_gather(dinv_v, [dq_v[sl]])
        nq_v[sl] = a * b
        return 0

    lax.fori_loop(0, QPT // LANES, normq_body, 0)
    pltpu.sync_copy(nq_v, normq_out.at[pl.ds(qbase, QPT)])

    gather_q(et_hbm, dq_v)
    pltpu.sync_copy(dq_v, etq_out.at[pl.ds(qbase, QPT)])


@functools.partial(
    pl.kernel,
    out_type=[
        jax.ShapeDtypeStruct((2 * NQHP, DP), jnp.float32),
        jax.ShapeDtypeStruct((2 * EHP,), jnp.float32),
        jax.ShapeDtypeStruct((2 * EHP,), jnp.float32),
    ],
    mesh=_MESH,
    compiler_params=_SC_PARAMS,
    scratch_types=[
        pltpu.VMEM_SHARED((EHP,), jnp.float32),
        pltpu.VMEM((QPT,), jnp.int32),
        pltpu.VMEM((QPT,), jnp.int32),
        pltpu.VMEM((QPT,), jnp.int32),
        pltpu.VMEM((QPT,), jnp.float32),
        pltpu.VMEM((DP, DP), jnp.float32),
        pltpu.VMEM((4, BS, DP), jnp.float32),
        pltpu.VMEM((4, BS, DP), jnp.float32),
        pltpu.VMEM((EPT,), jnp.float32),
        pltpu.VMEM((EPT,), jnp.float32),
        pltpu.VMEM((640,), jnp.float32),
        pltpu.SemaphoreType.DMA,
        pltpu.SemaphoreType.DMA,
        pltpu.SemaphoreType.DMA,
        pltpu.SemaphoreType.DMA,
        pltpu.SemaphoreType.DMA,
        pltpu.SemaphoreType.DMA,
        pltpu.SemaphoreType.DMA,
        pltpu.SemaphoreType.DMA,
    ],
)
def _sc_qualprep(h_hbm, relf_hbm, qent_hbm, qrel_hbm, qedge_hbm, norm_hbm,
                 qemb_out, s_out, scale_out,
                 s_sh, qent_v, qrel_v, qe_v, r_v, relf_t,
                 hring, qring, nrm_v, sl_v, zb_v,
                 sg0, sg1, sg2, sg3, sw0, sw1, sw2, sw3):
    c = lax.axis_index("c")
    t = lax.axis_index("s")
    qbase = c * NQHP + t * QPT
    ebase = c * EHP + t * EPT

    pltpu.sync_copy(qent_hbm.at[pl.ds(qbase, QPT)], qent_v)
    pltpu.sync_copy(qrel_hbm.at[pl.ds(qbase, QPT)], qrel_v)
    pltpu.sync_copy(qedge_hbm.at[pl.ds(qbase, QPT)], qe_v)
    pltpu.sync_copy(relf_hbm, relf_t)

    _fill(zb_v, 640, 0.0)
    for i in range(EPT // 640):
        pltpu.sync_copy(zb_v, s_sh.at[pl.ds(t * EPT + i * 640, 640)])
    plsc.subcore_barrier()

    j16 = lax.iota(jnp.int32, LANES)

    def compute(qb, sl):
        for k in range(BS // LANES):
            qr16 = qrel_v[pl.ds(qb + k * LANES, LANES)]
            rsum = jnp.zeros((LANES,), jnp.float32)
            for jj in range(LANES):
                j = k * LANES + jj
                qr_j = jnp.full((LANES,), qr16[jj], jnp.int32)
                racc = jnp.zeros((LANES,), jnp.float32)
                for kk in range(DP // LANES):
                    fs = pl.ds(kk * LANES, LANES)
                    rv = plsc.load_gather(relf_t, [qr_j, j16 + kk * LANES])
                    q = hring[sl, j, fs] * rv
                    qring[sl, j, fs] = q
                    racc = racc + q
                rsum = rsum + jnp.where(j16 == jj, jnp.sum(racc), 0.0)
            r_v[pl.ds(qb + k * LANES, LANES)] = rsum

    sgs = (sg0, sg1, sg2, sg3)
    sws = (sw0, sw1, sw2, sw3)
    nqb = QPT // BS

    for sl in range(4):
        pltpu.async_copy(h_hbm.at[qent_v.at[pl.ds(sl * BS, BS)]],
                         hring.at[sl], sgs[sl])

    def quad_body(q, _):
        ws = []
        for sl in range(4):
            b = (q * 4 + sl) * BS
            pltpu.make_async_copy(h_hbm.at[qent_v.at[pl.ds(b, BS)]],
                                  hring.at[sl], sgs[sl]).wait()
            compute(b, sl)
            ws.append(pltpu.async_copy(
                qring.at[sl], qemb_out.at[pl.ds(qbase + b, BS)], sws[sl]))

            @pl.when(q * 4 + sl + 4 < nqb)
            def _():
                bn = b + 4 * BS
                pltpu.async_copy(h_hbm.at[qent_v.at[pl.ds(bn, BS)]],
                                 hring.at[sl], sgs[sl])
        for w in ws:
            w.wait()
        return 0

    lax.fori_loop(0, nqb // 4, quad_body, 0)

    pltpu.sync_copy(r_v, s_sh.at[qe_v], add=True)
    plsc.subcore_barrier()
    pltpu.sync_copy(s_sh.at[pl.ds(t * EPT, EPT)], sl_v)
    pltpu.sync_copy(sl_v, s_out.at[pl.ds(ebase, EPT)])
    pltpu.sync_copy(norm_hbm.at[pl.ds(ebase, EPT)], nrm_v)

    def scale_body(i, _):
        sl = pl.ds(i * LANES, LANES)
        keep = sl_v[sl] == 0.0
        nrm_v[sl] = nrm_v[sl] * jnp.where(keep, 1.0, 0.0)
        return 0

    lax.fori_loop(0, EPT // LANES, scale_body, 0)
    pltpu.sync_copy(nrm_v, scale_out.at[pl.ds(ebase, EPT)])


def _make_sc_accum(cw):
    @functools.partial(
        pl.kernel,
        out_type=jax.ShapeDtypeStruct((2 * NR, cw), jnp.float32),
        mesh=_MESH,
        compiler_params=_SC_PARAMS,
        scratch_types=[
            pltpu.VMEM_SHARED((NR, cw), jnp.float32),
            pltpu.VMEM((DP, cw), jnp.float32),
            pltpu.VMEM((EBLK,), jnp.int32),
            pltpu.VMEM((EBLK,), jnp.int32),
            pltpu.VMEM((EBLK,), jnp.int32),
            pltpu.VMEM((EBLK,), jnp.float32),
            pltpu.VMEM((4, BS, cw), jnp.float32),
            pltpu.VMEM((4, BS, cw), jnp.float32),
            pltpu.VMEM((4, BS), jnp.int32),
            pltpu.VMEM((4, cw), jnp.float32),
            pltpu.VMEM((EBLK,), jnp.float32),
            pltpu.SemaphoreType.DMA,
            pltpu.SemaphoreType.DMA,
            pltpu.SemaphoreType.DMA,
            pltpu.SemaphoreType.DMA,
            pltpu.SemaphoreType.DMA,
            pltpu.SemaphoreType.DMA,
            pltpu.SemaphoreType.DMA,
            pltpu.SemaphoreType.DMA,
            pltpu.SemaphoreType.DMA,
            pltpu.SemaphoreType.DMA,
        ],
    )
    def _accum(h_hbm, relq_hbm, relf_hbm, dst_hbm, et_hbm, src_hbm,
               scale_hbm, s_hbm, srcq_hbm, dstq_hbm, etq_hbm, normq_hbm,
               qedge_hbm, qw_hbm,
               a_out,
               acc_sh, tbl_t, src_v, dst_v, et_v, scale_v,
               hring, mring, idxr, zrows, aux_v,
               sg0, sg1, sg2, sg3, ss0, ss1, ss2, ss3, sq0, sq1):
        c = lax.axis_index("c")
        t = lax.axis_index("s")
        ebase = c * EHP + t * EPT
        qbase = c * NQHP + t * QPT
        arow0 = t * ART
        sgs = (sg0, sg1, sg2, sg3)
        sss = (ss0, ss1, ss2, ss3)

        for i in range(4):
            for k in range(cw // LANES):
                zrows[i, pl.ds(k * LANES, LANES)] = jnp.zeros((LANES,),
                                                              jnp.float32)

        def zero_body(i, _):
            pltpu.sync_copy(zrows, acc_sh.at[pl.ds(arow0 + i * 4, 4)])
            return 0

        lax.fori_loop(0, ART // 4, zero_body, 0)
        pltpu.sync_copy(relq_hbm, tbl_t)
        plsc.subcore_barrier()

        j16 = lax.iota(jnp.int32, LANES)

        def compute_edge(b, sl):
            for k in range(BS // LANES):
                et16 = et_v[pl.ds(b + k * LANES, LANES)]
                sc16 = scale_v[pl.ds(b + k * LANES, LANES)]
                for jj in range(LANES):
                    j = k * LANES + jj
                    et_j = jnp.full((LANES,), et16[jj], jnp.int32)
                    scv = jnp.full((LANES,), sc16[jj], jnp.float32)
                    for kk in range(cw // LANES):
                        fs = pl.ds(kk * LANES, LANES)
                        rv = plsc.load_gather(tbl_t, [et_j, j16 + kk * LANES])
                        mring[sl, j, fs] = hring[sl, j, fs] * rv * scv

        nbb = EBLK // BS

        def blk_body(blk, _):
            bbase = ebase + blk * EBLK
            pltpu.sync_copy(src_hbm.at[pl.ds(bbase, EBLK)], src_v)
            pltpu.sync_copy(dst_hbm.at[pl.ds(bbase, EBLK)], dst_v)
            pltpu.sync_copy(et_hbm.at[pl.ds(bbase, EBLK)], et_v)
            pltpu.sync_copy(scale_hbm.at[pl.ds(bbase, EBLK)], scale_v)

            for sl in range(4):
                pltpu.async_copy(h_hbm.at[src_v.at[pl.ds(sl * BS, BS)]],
                                 hring.at[sl], sgs[sl])

            def quad_body(q, _):
                ss_list = []
                for sl in range(4):
                    b = (q * 4 + sl) * BS
                    pltpu.make_async_copy(
                        h_hbm.at[src_v.at[pl.ds(b, BS)]],
                        hring.at[sl], sgs[sl]).wait()
                    compute_edge(b, sl)
                    for _k in range(BS // LANES):
                        idxr[sl, pl.ds(_k * LANES, LANES)] = (
                            dst_v[pl.ds(b + _k * LANES, LANES)])
                    ss_list.append(pltpu.async_copy(
                        mring.at[sl], acc_sh.at[idxr.at[sl]], sss[sl],
                        add=True))

                    @pl.when(q * 4 + sl + 4 < nbb)
                    def _():
                        bn = b + 4 * BS
                        pltpu.async_copy(h_hbm.at[src_v.at[pl.ds(bn, BS)]],
                                         hring.at[sl], sgs[sl])
                for s in ss_list:
                    s.wait()
                return 0

            lax.fori_loop(0, nbb // 4, quad_body, 0)
            return 0

        lax.fori_loop(0, EPT // EBLK, blk_body, 0)

        pltpu.sync_copy(qedge_hbm.at[pl.ds(qbase, QPT)], src_v)

        def glob_body(i, _):
            sl = pl.ds(i * LANES, LANES)
            src_v[sl] = src_v[sl] + c * EHP
            return 0

        lax.fori_loop(0, QPT // LANES, glob_body, 0)
        pltpu.sync_copy(s_hbm.at[src_v], aux_v)
        pltpu.sync_copy(normq_hbm.at[pl.ds(qbase, QPT)], scale_v)

        def scaleq_body(i, _):
            sl = pl.ds(i * LANES, LANES)
            keep = aux_v[sl] != 0.0
            scale_v[sl] = scale_v[sl] * jnp.where(keep, 1.0, 0.0)
            return 0

        lax.fori_loop(0, QPT // LANES, scaleq_body, 0)
        pltpu.sync_copy(srcq_hbm.at[pl.ds(qbase, QPT)], src_v)
        pltpu.sync_copy(dstq_hbm.at[pl.ds(qbase, QPT)], dst_v)
        pltpu.sync_copy(etq_hbm.at[pl.ds(qbase, QPT)], et_v)
        pltpu.sync_copy(relf_hbm, tbl_t)

        def compute_qual(b, slh, slw):
            for k in range(BS // LANES):
                et16 = et_v[pl.ds(b + k * LANES, LANES)]
                sc16 = scale_v[pl.ds(b + k * LANES, LANES)]
                for jj in range(LANES):
                    j = k * LANES + jj
                    et_j = jnp.full((LANES,), et16[jj], jnp.int32)
                    scv = jnp.full((LANES,), sc16[jj], jnp.float32)
                    for kk in range(cw // LANES):
                        fs = pl.ds(kk * LANES, LANES)
                        rv = plsc.load_gather(tbl_t, [et_j, j16 + kk * LANES])
                        mring[slh, j, fs] = (hring[slh, j, fs] * rv *
                                             hring[slw, j, fs] * scv)

        def qpair_body(p, _):
            b0 = p * 2 * BS
            b1 = b0 + BS
            g0 = pltpu.async_copy(h_hbm.at[src_v.at[pl.ds(b0, BS)]],
                                  hring.at[0], sg0)
            q0 = pltpu.async_copy(qw_hbm.at[pl.ds(qbase + b0, BS)],
                                  hring.at[2], sq0)
            g1 = pltpu.async_copy(h_hbm.at[src_v.at[pl.ds(b1, BS)]],
                                  hring.at[1], sg1)
            q1 = pltpu.async_copy(qw_hbm.at[pl.ds(qbase + b1, BS)],
                                  hring.at[3], sq1)
            g0.wait()
            q0.wait()
            compute_qual(b0, 0, 2)
            for _k in range(BS // LANES):
                idxr[0, pl.ds(_k * LANES, LANES)] = (
                    dst_v[pl.ds(b0 + _k * LANES, LANES)])
            s0 = pltpu.async_copy(mring.at[0], acc_sh.at[idxr.at[0]], ss0,
                                  add=True)
            g1.wait()
            q1.wait()
            compute_qual(b1, 1, 3)
            for _k in range(BS // LANES):
                idxr[1, pl.ds(_k * LANES, LANES)] = (
                    dst_v[pl.ds(b1 + _k * LANES, LANES)])
            s1 = pltpu.async_copy(mring.at[1], acc_sh.at[idxr.at[1]], ss1,
                                  add=True)
            s0.wait()
            s1.wait()
            return 0

        lax.fori_loop(0, QPT // (2 * BS), qpair_body, 0)
        plsc.subcore_barrier()
        pltpu.sync_copy(acc_sh.at[pl.ds(arow0, ART)],
                        a_out.at[pl.ds(c * NR + arow0, ART)])

    return _accum


_SC_ACCUM_A = _make_sc_accum(CW_A)
_SC_ACCUM_B = _make_sc_accum(CW_B)


def kernel(x, edge_index, edge_type, qualifier_index, rel_embs, fr_W, fr_b,
           w_in_0, w_out_0, w_loop_0, w_rel_0, w_q_0, loop_rel_0,
           w_in_1, w_out_1, w_loop_1, w_rel_1, w_q_1, loop_rel_1):
    pad_node = DUMMY + (jnp.arange(EHP - EH, dtype=jnp.int32) % 8)
    pad2 = jnp.broadcast_to(pad_node, (2, EHP - EH))

    def pad_edges(a, pad):
        return jnp.concatenate([a.reshape(2, EH), pad], axis=1).reshape(-1)

    src_p = pad_edges(edge_index[0], pad2)
    dst_p = pad_edges(edge_index[1], pad2)
    et_p = pad_edges(edge_type, jnp.zeros((2, EHP - EH), jnp.int32))

    pad_qn = DUMMY + (jnp.arange(NQHP - NQH, dtype=jnp.int32) % 8)
    pad_q2 = jnp.broadcast_to(pad_qn, (2, NQHP - NQH))
    pad_qe = EH + (jnp.arange(NQHP - NQH, dtype=jnp.int32) % 8)
    pad_qe2 = jnp.broadcast_to(pad_qe, (2, NQHP - NQH))

    def pad_quals(a, pad):
        return jnp.concatenate([a.reshape(2, NQH), pad], axis=1).reshape(-1)

    qrel_p = pad_quals(qualifier_index[0], jnp.zeros((2, NQHP - NQH), jnp.int32))
    qent_p = pad_quals(qualifier_index[1], pad_q2)
    qedge_p = pad_quals(qualifier_index[2], pad_qe2)

    norm, src_q, dst_q, et_q, norm_q = _sc_prep(src_p, dst_p, et_p, qedge_p)

    x_p = jnp.pad(x, ((0, NR - N), (0, 0)))
    h = _tc_matmul_bias(x_p, _pad_d(fr_W, (1,)), _pad_d(fr_b, (0,)), bm=1256)

    rel = _pad_d(rel_embs, (1,))
    layers = [
        (w_in_0, w_out_0, w_loop_0, w_rel_0, w_q_0, loop_rel_0),
        (w_in_1, w_out_1, w_loop_1, w_rel_1, w_q_1, loop_rel_1),
    ]
    for w_in, w_out, w_loop, w_rel, w_q, loop_rel in layers:
        w_in, w_out, w_loop, w_rel, w_q = (
            _pad_d(w, (0, 1)) for w in (w_in, w_out, w_loop, w_rel, w_q))
        loop_rel = _pad_d(loop_rel, (1,))
        rel_full = jnp.zeros((DP, DP), jnp.float32)
        rel_full = rel_full.at[:R].set(rel).at[R].set(loop_rel[0])
        relq, rel_next = _tc_relprep(rel_full, w_q, w_rel)

        q_emb, s, scale = _sc_qualprep(h, rel_full, qent_p, qrel_p, qedge_p,
                                       norm)
        qw = _tc_qw(q_emb, w_q, bm=1280)

        args = (dst_p, et_p, src_p, scale, s, src_q, dst_q, et_q, norm_q,
                qedge_p)
        acc_a = _SC_ACCUM_A(h[:, :CW_A], relq[:, :CW_A], rel_full[:, :CW_A],
                            *args, qw[:, :CW_A])
        acc_b = _SC_ACCUM_B(h[:, CW_A:], relq[:, CW_A:], rel_full[:, CW_A:],
                            *args, qw[:, CW_A:])

        h = _tc_combine(acc_a[:NR], acc_b[:NR], acc_a[NR:], acc_b[NR:], h,
                        w_in, w_out, w_loop, loop_rel, bm=1256)
        rel = rel_next[:R]

    return h[:N, :D], rel[:R, :D]


# speedup vs baseline: 3.4781x; 1.2442x over previous
import functools

import jax
import jax.numpy as jnp
from jax import lax
from jax.experimental import pallas as pl
from jax.experimental.pallas import tpu as pltpu
from jax.experimental.pallas import tpu_sc as plsc

N = 10000
FEAT = 1024
D = 200
DP = 208
R = 200
E = 160000
EH = E // 2
NQH = 40000 // 2

NS = 16
LANES = 16
BS = 16

NR = 10048
DUMMY = 10016
EHP = 81920
EPT = EHP // NS
EBLK = 1280
NQHP = 20480
QPT = NQHP // NS
ART = NR // NS
WACC = 224

_MESH = plsc.VectorSubcoreMesh(core_axis_name="c", subcore_axis_name="s")
_SC_PARAMS = pltpu.CompilerParams(use_tc_tiling_on_sc=False,
                                  needs_layout_passes=False)


def _pad_d(a, axes):
    pads = [(0, 0)] * a.ndim
    for ax in axes:
        pads[ax] = (0, DP - a.shape[ax])
    return jnp.pad(a, pads)


def _mm_bias_body(x_ref, w_ref, b_ref, o_ref):
    o_ref[...] = jnp.dot(x_ref[...], w_ref[...],
                         preferred_element_type=jnp.float32) + b_ref[...]


def _tc_matmul_bias(x, w, b, bm):
    m, k = x.shape
    n = w.shape[1]
    return pl.pallas_call(
        _mm_bias_body,
        grid=(m // bm,),
        in_specs=[
            pl.BlockSpec((bm, k), lambda i: (i, 0)),
            pl.BlockSpec((k, n), lambda i: (0, 0)),
            pl.BlockSpec((1, n), lambda i: (0, 0)),
        ],
        out_specs=pl.BlockSpec((bm, n), lambda i: (i, 0)),
        out_shape=jax.ShapeDtypeStruct((m, n), jnp.float32),
    )(x, w, b.reshape(1, n))


def _qw_body(q_ref, w_ref, o_ref):
    o_ref[...] = jnp.dot(q_ref[...], w_ref[...],
                         preferred_element_type=jnp.float32)


def _tc_qw(q_emb, w_q, bm):
    m = q_emb.shape[0]
    return pl.pallas_call(
        _qw_body,
        grid=(m // bm,),
        in_specs=[
            pl.BlockSpec((bm, DP), lambda i: (i, 0)),
            pl.BlockSpec((DP, DP), lambda i: (0, 0)),
        ],
        out_specs=pl.BlockSpec((bm, DP), lambda i: (i, 0)),
        out_shape=jax.ShapeDtypeStruct((m, DP), jnp.float32),
    )(q_emb, w_q)


def _relprep_body(rf_ref, wq_ref, wr_ref, relq_ref, reln_ref):
    wq = wq_ref[...]
    cq = jnp.sum(wq, axis=0, keepdims=True)
    relq_ref[...] = rf_ref[...] * cq
    reln_ref[...] = jnp.dot(rf_ref[...], wr_ref[...],
                            preferred_element_type=jnp.float32)


def _tc_relprep(rel_full, w_q, w_rel):
    return pl.pallas_call(
        _relprep_body,
        in_specs=[pl.BlockSpec((DP, DP), lambda: (0, 0))] * 3,
        out_specs=[pl.BlockSpec((DP, DP), lambda: (0, 0))] * 2,
        out_shape=[jax.ShapeDtypeStruct((DP, DP), jnp.float32)] * 2,
        grid=(),
    )(rel_full, w_q, w_rel)


def _combine_body(ain_ref, aout_ref, h_ref, wi_ref, wo_ref, wl_ref, lr_ref,
                  o_ref):
    f32 = jnp.float32
    acc = jnp.dot(ain_ref[...], wi_ref[...], preferred_element_type=f32)
    acc += jnp.dot(aout_ref[...], wo_ref[...], preferred_element_type=f32)
    acc += jnp.dot(h_ref[...] * lr_ref[...], wl_ref[...],
                   preferred_element_type=f32)
    o_ref[...] = jnp.tanh(acc * (1.0 / 3.0))


def _tc_combine(a_in, a_out, h, w_in, w_out, w_loop, loop_rel, bm):
    m = h.shape[0]
    return pl.pallas_call(
        _combine_body,
        grid=(m // bm,),
        in_specs=[
            pl.BlockSpec((bm, WACC), lambda i: (i, 0)),
            pl.BlockSpec((bm, WACC), lambda i: (i, 0)),
            pl.BlockSpec((bm, DP), lambda i: (i, 0)),
            pl.BlockSpec((WACC, DP), lambda i: (0, 0)),
            pl.BlockSpec((WACC, DP), lambda i: (0, 0)),
            pl.BlockSpec((DP, DP), lambda i: (0, 0)),
            pl.BlockSpec((1, DP), lambda i: (0, 0)),
        ],
        out_specs=pl.BlockSpec((bm, DP), lambda i: (i, 0)),
        out_shape=jax.ShapeDtypeStruct((m, DP), jnp.float32),
    )(a_in, a_out, h, w_in, w_out, w_loop, loop_rel)


def _rsqrt16(d):
    i = plsc.bitcast(d, jnp.int32)
    i = jnp.int32(0x5F3759DF) - lax.shift_right_logical(i, 1)
    y = plsc.bitcast(i, jnp.float32)
    for _ in range(3):
        y = y * (1.5 - 0.5 * d * y * y)
    return jnp.where(d > 0, y, 0.0)


def _fill(ref, n, value):
    v = jnp.full((LANES,), value, ref.dtype)

    def body(i, _):
        ref[pl.ds(i * LANES, LANES)] = v
        return 0

    lax.fori_loop(0, n // LANES, body, 0)


@functools.partial(
    pl.kernel,
    out_type=[
        jax.ShapeDtypeStruct((2 * EHP,), jnp.float32),
        jax.ShapeDtypeStruct((2 * NQHP,), jnp.int32),
        jax.ShapeDtypeStruct((2 * NQHP,), jnp.int32),
        jax.ShapeDtypeStruct((2 * NQHP,), jnp.int32),
        jax.ShapeDtypeStruct((2 * NQHP,), jnp.float32),
    ],
    mesh=_MESH,
    compiler_params=_SC_PARAMS,
    scratch_types=[
        pltpu.VMEM_SHARED((NR,), jnp.float32),
        pltpu.VMEM((EPT,), jnp.int32),
        pltpu.VMEM((EPT,), jnp.int32),
        pltpu.VMEM((EPT,), jnp.float32),
        pltpu.VMEM((640,), jnp.float32),
        pltpu.VMEM((NR,), jnp.float32),
        pltpu.VMEM((EHP,), jnp.int32),
        pltpu.VMEM((QPT,), jnp.int32),
        pltpu.VMEM((QPT,), jnp.int32),
        pltpu.VMEM((QPT,), jnp.int32),
        pltpu.VMEM((QPT,), jnp.float32),
    ],
)
def _sc_prep(src_hbm, dst_hbm, et_hbm, qedge_hbm,
             norm_out, srcq_out, dstq_out, etq_out, normq_out,
             deg_sh, src_v, dst_v, val_v, zb_v, dinv_v, tbl_v,
             qe_v, sq_v, dq_v, nq_v):
    c = lax.axis_index("c")
    t = lax.axis_index("s")
    ebase = c * EHP + t * EPT
    qbase = c * NQHP + t * QPT

    pltpu.sync_copy(src_hbm.at[pl.ds(ebase, EPT)], src_v)
    pltpu.sync_copy(dst_hbm.at[pl.ds(ebase, EPT)], dst_v)

    _fill(zb_v, 640, 0.0)
    zbase = jnp.minimum(t * 640, NR - 640)
    pltpu.sync_copy(zb_v, deg_sh.at[pl.ds(zbase, 640)])
    _fill(val_v, EPT, 1.0)
    plsc.subcore_barrier()

    pltpu.sync_copy(val_v, deg_sh.at[src_v], add=True)
    plsc.subcore_barrier()

    pltpu.sync_copy(deg_sh, dinv_v)

    def inv_body(i, _):
        d = dinv_v[pl.ds(i * LANES, LANES)]
        dinv_v[pl.ds(i * LANES, LANES)] = _rsqrt16(d)
        return 0

    lax.fori_loop(0, NR // LANES, inv_body, 0)

    def norm_body(i, _):
        sl = pl.ds(i * LANES, LANES)
        a = plsc.load_gather(dinv_v, [src_v[sl]])
        b = plsc.load_gather(dinv_v, [dst_v[sl]])
        val_v[sl] = a * b
        return 0

    lax.fori_loop(0, EPT // LANES, norm_body, 0)
    pltpu.sync_copy(val_v, norm_out.at[pl.ds(ebase, EPT)])

    pltpu.sync_copy(qedge_hbm.at[pl.ds(qbase, QPT)], qe_v)

    def gather_q(table_hbm, out_v):
        pltpu.sync_copy(table_hbm.at[pl.ds(c * EHP, EHP)], tbl_v)

        def body(i, _):
            sl = pl.ds(i * LANES, LANES)
            out_v[sl] = plsc.load_gather(tbl_v, [qe_v[sl]])
            return 0

        lax.fori_loop(0, QPT // LANES, body, 0)

    gather_q(src_hbm, sq_v)
    pltpu.sync_copy(sq_v, srcq_out.at[pl.ds(qbase, QPT)])
    gather_q(dst_hbm, dq_v)
    pltpu.sync_copy(dq_v, dstq_out.at[pl.ds(qbase, QPT)])

    def normq_body(i, _):
        sl = pl.ds(i * LANES, LANES)
        a = plsc.load_gather(dinv_v, [sq_v[sl]])
        b = plsc.load_gather(dinv_v, [dq_v[sl]])
        nq_v[sl] = a * b
        return 0

    lax.fori_loop(0, QPT // LANES, normq_body, 0)
    pltpu.sync_copy(nq_v, normq_out.at[pl.ds(qbase, QPT)])

    gather_q(et_hbm, dq_v)
    pltpu.sync_copy(dq_v, etq_out.at[pl.ds(qbase, QPT)])


@functools.partial(
    pl.kernel,
    out_type=[
        jax.ShapeDtypeStruct((2 * NQHP, DP), jnp.float32),
        jax.ShapeDtypeStruct((2 * EHP,), jnp.float32),
        jax.ShapeDtypeStruct((2 * EHP,), jnp.float32),
    ],
    mesh=_MESH,
    compiler_params=_SC_PARAMS,
    scratch_types=[
        pltpu.VMEM_SHARED((EHP,), jnp.float32),
        pltpu.VMEM((QPT,), jnp.int32),
        pltpu.VMEM((QPT,), jnp.int32),
        pltpu.VMEM((QPT,), jnp.int32),
        pltpu.VMEM((QPT,), jnp.float32),
        pltpu.VMEM((DP, DP), jnp.float32),
        pltpu.VMEM((4, BS, DP), jnp.float32),
        pltpu.VMEM((4, BS, DP), jnp.float32),
        pltpu.VMEM((EPT,), jnp.float32),
        pltpu.VMEM((EPT,), jnp.float32),
        pltpu.VMEM((640,), jnp.float32),
        pltpu.SemaphoreType.DMA,
        pltpu.SemaphoreType.DMA,
        pltpu.SemaphoreType.DMA,
        pltpu.SemaphoreType.DMA,
        pltpu.SemaphoreType.DMA,
        pltpu.SemaphoreType.DMA,
        pltpu.SemaphoreType.DMA,
        pltpu.SemaphoreType.DMA,
    ],
)
def _sc_qualprep(h_hbm, relf_hbm, qent_hbm, qrel_hbm, qedge_hbm, norm_hbm,
                 qemb_out, s_out, scale_out,
                 s_sh, qent_v, qrel_v, qe_v, r_v, relf_t,
                 hring, qring, nrm_v, sl_v, zb_v,
                 sg0, sg1, sg2, sg3, sw0, sw1, sw2, sw3):
    c = lax.axis_index("c")
    t = lax.axis_index("s")
    qbase = c * NQHP + t * QPT
    ebase = c * EHP + t * EPT

    pltpu.sync_copy(qent_hbm.at[pl.ds(qbase, QPT)], qent_v)
    pltpu.sync_copy(qrel_hbm.at[pl.ds(qbase, QPT)], qrel_v)
    pltpu.sync_copy(qedge_hbm.at[pl.ds(qbase, QPT)], qe_v)
    pltpu.sync_copy(relf_hbm, relf_t)

    _fill(zb_v, 640, 0.0)
    for i in range(EPT // 640):
        pltpu.sync_copy(zb_v, s_sh.at[pl.ds(t * EPT + i * 640, 640)])
    plsc.subcore_barrier()

    j16 = lax.iota(jnp.int32, LANES)

    def compute(qb, sl):
        for k in range(BS // LANES):
            qr16 = qrel_v[pl.ds(qb + k * LANES, LANES)]
            rsum = jnp.zeros((LANES,), jnp.float32)
            for jj in range(LANES):
                j = k * LANES + jj
                qr_j = jnp.full((LANES,), qr16[jj], jnp.int32)
                racc = jnp.zeros((LANES,), jnp.float32)
                for kk in range(DP // LANES):
                    fs = pl.ds(kk * LANES, LANES)
                    rv = plsc.load_gather(relf_t, [qr_j, j16 + kk * LANES])
                    q = hring[sl, j, fs] * rv
                    qring[sl, j, fs] = q
                    racc = racc + q
                rsum = rsum + jnp.where(j16 == jj, jnp.sum(racc), 0.0)
            r_v[pl.ds(qb + k * LANES, LANES)] = rsum

    sgs = (sg0, sg1, sg2, sg3)
    sws = (sw0, sw1, sw2, sw3)
    nqb = QPT // BS

    for sl in range(4):
        pltpu.async_copy(h_hbm.at[qent_v.at[pl.ds(sl * BS, BS)]],
                         hring.at[sl], sgs[sl])

    def quad_body(q, _):
        ws = []
        for sl in range(4):
            b = (q * 4 + sl) * BS
            pltpu.make_async_copy(h_hbm.at[qent_v.at[pl.ds(b, BS)]],
                                  hring.at[sl], sgs[sl]).wait()
            compute(b, sl)
            ws.append(pltpu.async_copy(
                qring.at[sl], qemb_out.at[pl.ds(qbase + b, BS)], sws[sl]))

            @pl.when(q * 4 + sl + 4 < nqb)
            def _():
                bn = b + 4 * BS
                pltpu.async_copy(h_hbm.at[qent_v.at[pl.ds(bn, BS)]],
                                 hring.at[sl], sgs[sl])
        for w in ws:
            w.wait()
        return 0

    lax.fori_loop(0, nqb // 4, quad_body, 0)

    pltpu.sync_copy(r_v, s_sh.at[qe_v], add=True)
    plsc.subcore_barrier()
    pltpu.sync_copy(s_sh.at[pl.ds(t * EPT, EPT)], sl_v)
    pltpu.sync_copy(sl_v, s_out.at[pl.ds(ebase, EPT)])
    pltpu.sync_copy(norm_hbm.at[pl.ds(ebase, EPT)], nrm_v)

    def scale_body(i, _):
        sl = pl.ds(i * LANES, LANES)
        keep = sl_v[sl] == 0.0
        nrm_v[sl] = nrm_v[sl] * jnp.where(keep, 1.0, 0.0)
        return 0

    lax.fori_loop(0, EPT // LANES, scale_body, 0)
    pltpu.sync_copy(nrm_v, scale_out.at[pl.ds(ebase, EPT)])


@functools.partial(
    pl.kernel,
    out_type=jax.ShapeDtypeStruct((2 * NR, WACC), jnp.bfloat16),
    mesh=_MESH,
    compiler_params=_SC_PARAMS,
    scratch_types=[
        pltpu.VMEM_SHARED((NR, WACC), jnp.bfloat16),
        pltpu.VMEM((DP * WACC,), jnp.bfloat16),
        pltpu.VMEM((EBLK,), jnp.int32),
        pltpu.VMEM((EBLK,), jnp.int32),
        pltpu.VMEM((EBLK,), jnp.int32),
        pltpu.VMEM((EBLK,), jnp.float32),
        pltpu.VMEM((4, BS, WACC), jnp.bfloat16),
        pltpu.VMEM((4, BS, WACC), jnp.bfloat16),
        pltpu.VMEM((2, BS, WACC), jnp.bfloat16),
        pltpu.VMEM((4, BS), jnp.int32),
        pltpu.VMEM((4, WACC), jnp.bfloat16),
        pltpu.VMEM((EBLK,), jnp.float32),
        pltpu.SemaphoreType.DMA,
        pltpu.SemaphoreType.DMA,
        pltpu.SemaphoreType.DMA,
        pltpu.SemaphoreType.DMA,
        pltpu.SemaphoreType.DMA,
        pltpu.SemaphoreType.DMA,
        pltpu.SemaphoreType.DMA,
        pltpu.SemaphoreType.DMA,
        pltpu.SemaphoreType.DMA,
        pltpu.SemaphoreType.DMA,
    ],
)
def _sc_accum(h_hbm, relq_hbm, relf_hbm, dst_hbm, et_hbm, src_hbm,
              scale_hbm, s_hbm, srcq_hbm, dstq_hbm, etq_hbm, normq_hbm,
              qedge_hbm, qw_hbm,
              a_out,
              acc_sh, tbl_t, src_v, dst_v, et_v, scale_v,
              hring, mring, qwring, idxr, zrows, aux_v,
              sg0, sg1, sg2, sg3, ss0, ss1, ss2, ss3, sq0, sq1):
    c = lax.axis_index("c")
    t = lax.axis_index("s")
    ebase = c * EHP + t * EPT
    qbase = c * NQHP + t * QPT
    arow0 = t * ART
    sgs = (sg0, sg1, sg2, sg3)
    sss = (ss0, ss1, ss2, ss3)
    bzero = jnp.zeros((2 * LANES,), jnp.bfloat16)

    for i in range(4):
        for k in range(WACC // (2 * LANES)):
            zrows[i, pl.ds(k * 2 * LANES, 2 * LANES)] = bzero

    def zero_body(i, _):
        pltpu.sync_copy(zrows, acc_sh.at[pl.ds(arow0 + i * 4, 4)])
        return 0

    lax.fori_loop(0, ART // 4, zero_body, 0)
    pltpu.sync_copy(relq_hbm, tbl_t)
    plsc.subcore_barrier()

    def compute_edge(b, sl):
        for k in range(BS // LANES):
            et16 = et_v[pl.ds(b + k * LANES, LANES)]
            sc16 = scale_v[pl.ds(b + k * LANES, LANES)]
            for jj in range(LANES):
                j = k * LANES + jj
                base = et16[jj] * WACC
                scf = jnp.full((LANES,), sc16[jj], jnp.float32)
                scv = plsc.pack(scf, scf, format=plsc.PackFormat.INTERLEAVED)
                for kk in range(WACC // (2 * LANES)):
                    fs = pl.ds(kk * 2 * LANES, 2 * LANES)
                    rv = tbl_t[pl.ds(base + kk * 2 * LANES, 2 * LANES)]
                    mring[sl, j, fs] = hring[sl, j, fs] * rv * scv

    nbb = EBLK // BS

    def blk_body(blk, _):
        bbase = ebase + blk * EBLK
        pltpu.sync_copy(src_hbm.at[pl.ds(bbase, EBLK)], src_v)
        pltpu.sync_copy(dst_hbm.at[pl.ds(bbase, EBLK)], dst_v)
        pltpu.sync_copy(et_hbm.at[pl.ds(bbase, EBLK)], et_v)
        pltpu.sync_copy(scale_hbm.at[pl.ds(bbase, EBLK)], scale_v)

        for sl in range(4):
            pltpu.async_copy(h_hbm.at[src_v.at[pl.ds(sl * BS, BS)]],
                             hring.at[sl], sgs[sl])

        def quad_body(q, _):
            ss_list = []
            for sl in range(4):
                b = (q * 4 + sl) * BS
                pltpu.make_async_copy(
                    h_hbm.at[src_v.at[pl.ds(b, BS)]],
                    hring.at[sl], sgs[sl]).wait()
                compute_edge(b, sl)
                idxr[sl, pl.ds(0, LANES)] = dst_v[pl.ds(b, LANES)]
                ss_list.append(pltpu.async_copy(
                    mring.at[sl], acc_sh.at[idxr.at[sl]], sss[sl],
                    add=True))

                @pl.when(q * 4 + sl + 4 < nbb)
                def _():
                    bn = b + 4 * BS
                    pltpu.async_copy(h_hbm.at[src_v.at[pl.ds(bn, BS)]],
                                     hring.at[sl], sgs[sl])
            for s in ss_list:
                s.wait()
            return 0

        lax.fori_loop(0, nbb // 4, quad_body, 0)
        return 0

    lax.fori_loop(0, EPT // EBLK, blk_body, 0)

    pltpu.sync_copy(qedge_hbm.at[pl.ds(qbase, QPT)], src_v)

    def glob_body(i, _):
        sl = pl.ds(i * LANES, LANES)
        src_v[sl] = src_v[sl] + c * EHP
        return 0

    lax.fori_loop(0, QPT // LANES, glob_body, 0)
    pltpu.sync_copy(s_hbm.at[src_v], aux_v)
    pltpu.sync_copy(normq_hbm.at[pl.ds(qbase, QPT)], scale_v)

    def scaleq_body(i, _):
        sl = pl.ds(i * LANES, LANES)
        keep = aux_v[sl] != 0.0
        scale_v[sl] = scale_v[sl] * jnp.where(keep, 1.0, 0.0)
        return 0

    lax.fori_loop(0, QPT // LANES, scaleq_body, 0)
    pltpu.sync_copy(srcq_hbm.at[pl.ds(qbase, QPT)], src_v)
    pltpu.sync_copy(dstq_hbm.at[pl.ds(qbase, QPT)], dst_v)
    pltpu.sync_copy(etq_hbm.at[pl.ds(qbase, QPT)], et_v)
    pltpu.sync_copy(relf_hbm, tbl_t)

    def compute_qual(b, slh):
        for k in range(BS // LANES):
            et16 = et_v[pl.ds(b + k * LANES, LANES)]
            sc16 = scale_v[pl.ds(b + k * LANES, LANES)]
            for jj in range(LANES):
                j = k * LANES + jj
                base = et16[jj] * WACC
                scf = jnp.full((LANES,), sc16[jj], jnp.float32)
                scv = plsc.pack(scf, scf, format=plsc.PackFormat.INTERLEAVED)
                for kk in range(WACC // (2 * LANES)):
                    fs = pl.ds(kk * 2 * LANES, 2 * LANES)
                    rv = tbl_t[pl.ds(base + kk * 2 * LANES, 2 * LANES)]
                    mring[slh, j, fs] = (hring[slh, j, fs] * rv *
                                         qwring[slh, j, fs] * scv)

    def qpair_body(p, _):
        b0 = p * 2 * BS
        b1 = b0 + BS
        g0 = pltpu.async_copy(h_hbm.at[src_v.at[pl.ds(b0, BS)]],
                              hring.at[0], sg0)
        q0 = pltpu.async_copy(qw_hbm.at[pl.ds(qbase + b0, BS)],
                              qwring.at[0], sq0)
        g1 = pltpu.async_copy(h_hbm.at[src_v.at[pl.ds(b1, BS)]],
                              hring.at[1], sg1)
        q1 = pltpu.async_copy(qw_hbm.at[pl.ds(qbase + b1, BS)],
                              qwring.at[1], sq1)
        g0.wait()
        q0.wait()
        compute_qual(b0, 0)
        idxr[0, pl.ds(0, LANES)] = dst_v[pl.ds(b0, LANES)]
        s0 = pltpu.async_copy(mring.at[0], acc_sh.at[idxr.at[0]], ss0,
                              add=True)
        g1.wait()
        q1.wait()
        compute_qual(b1, 1)
        idxr[1, pl.ds(0, LANES)] = dst_v[pl.ds(b1, LANES)]
        s1 = pltpu.async_copy(mring.at[1], acc_sh.at[idxr.at[1]], ss1,
                              add=True)
        s0.wait()
        s1.wait()
        return 0

    lax.fori_loop(0, QPT // (2 * BS), qpair_body, 0)
    plsc.subcore_barrier()
    pltpu.sync_copy(acc_sh.at[pl.ds(arow0, ART)],
                    a_out.at[pl.ds(c * NR + arow0, ART)])


def kernel(x, edge_index, edge_type, qualifier_index, rel_embs, fr_W, fr_b,
           w_in_0, w_out_0, w_loop_0, w_rel_0, w_q_0, loop_rel_0,
           w_in_1, w_out_1, w_loop_1, w_rel_1, w_q_1, loop_rel_1):
    pad_node = DUMMY + (jnp.arange(EHP - EH, dtype=jnp.int32) % 8)
    pad2 = jnp.broadcast_to(pad_node, (2, EHP - EH))

    def pad_edges(a, pad):
        return jnp.concatenate([a.reshape(2, EH), pad], axis=1).reshape(-1)

    src_p = pad_edges(edge_index[0], pad2)
    dst_p = pad_edges(edge_index[1], pad2)
    et_p = pad_edges(edge_type, jnp.zeros((2, EHP - EH), jnp.int32))

    pad_qn = DUMMY + (jnp.arange(NQHP - NQH, dtype=jnp.int32) % 8)
    pad_q2 = jnp.broadcast_to(pad_qn, (2, NQHP - NQH))
    pad_qe = EH + (jnp.arange(NQHP - NQH, dtype=jnp.int32) % 8)
    pad_qe2 = jnp.broadcast_to(pad_qe, (2, NQHP - NQH))

    def pad_quals(a, pad):
        return jnp.concatenate([a.reshape(2, NQH), pad], axis=1).reshape(-1)

    qrel_p = pad_quals(qualifier_index[0], jnp.zeros((2, NQHP - NQH), jnp.int32))
    qent_p = pad_quals(qualifier_index[1], pad_q2)
    qedge_p = pad_quals(qualifier_index[2], pad_qe2)

    norm, src_q, dst_q, et_q, norm_q = _sc_prep(src_p, dst_p, et_p, qedge_p)

    x_p = jnp.pad(x, ((0, NR - N), (0, 0)))
    h = _tc_matmul_bias(x_p, _pad_d(fr_W, (1,)), _pad_d(fr_b, (0,)), bm=1256)

    rel = _pad_d(rel_embs, (1,))
    layers = [
        (w_in_0, w_out_0, w_loop_0, w_rel_0, w_q_0, loop_rel_0),
        (w_in_1, w_out_1, w_loop_1, w_rel_1, w_q_1, loop_rel_1),
    ]
    for w_in, w_out, w_loop, w_rel, w_q, loop_rel in layers:
        w_in, w_out, w_loop, w_rel, w_q = (
            _pad_d(w, (0, 1)) for w in (w_in, w_out, w_loop, w_rel, w_q))
        loop_rel = _pad_d(loop_rel, (1,))
        rel_full = jnp.zeros((DP, DP), jnp.float32)
        rel_full = rel_full.at[:R].set(rel).at[R].set(loop_rel[0])
        relq, rel_next = _tc_relprep(rel_full, w_q, w_rel)

        q_emb, s, scale = _sc_qualprep(h, rel_full, qent_p, qrel_p, qedge_p,
                                       norm)
        qw = _tc_qw(q_emb, w_q, bm=1280)

        bf = jnp.bfloat16
        padw = ((0, 0), (0, WACC - DP))
        h_bf = jnp.pad(h, padw).astype(bf)
        relq_bf = jnp.pad(relq, padw).astype(bf).reshape(-1)
        relf_bf = jnp.pad(rel_full, padw).astype(bf).reshape(-1)
        qw_bf = jnp.pad(qw, padw).astype(bf)
        acc = _sc_accum(h_bf, relq_bf, relf_bf, dst_p, et_p, src_p, scale, s,
                        src_q, dst_q, et_q, norm_q, qedge_p, qw_bf)

        padr = ((0, WACC - DP), (0, 0))
        h = _tc_combine(acc[:NR], acc[NR:], h, jnp.pad(w_in, padr),
                        jnp.pad(w_out, padr), w_loop, loop_rel, bm=1256)
        rel = rel_next[:R]

    return h[:N, :D], rel[:R, :D]
